# baseline (speedup 1.0000x reference)
import functools

import jax
import jax.numpy as jnp
from jax import lax
from jax.experimental import pallas as pl
from jax.experimental.pallas import tpu as pltpu
from jax.experimental.pallas import tpu_sc as plsc

SLOPE = 0.2
NC = 2
NS = 16
NW = NC * NS
CG = 128
CGD = 80
RB = 1000
EB = 2048
F32 = jnp.float32


def _lrelu(x):
    return jnp.where(x >= 0, x, SLOPE * x)


def _mesh():
    return plsc.VectorSubcoreMesh(core_axis_name="c", subcore_axis_name="s",
                                  num_cores=NC, num_subcores=NS)


def _sc_deg(row3d, n_ent):
    nch = row3d.shape[1]
    cg = row3d.shape[2]
    nz = n_ent // CGD

    @functools.partial(
        pl.kernel,
        out_type=jax.ShapeDtypeStruct((NC * n_ent,), F32),
        mesh=_mesh(),
        scratch_types=[
            pltpu.VMEM((nch, cg), jnp.int32),
            pltpu.VMEM((cg,), F32),
            pltpu.VMEM((CGD,), F32),
            pltpu.VMEM_SHARED((n_ent,), F32),
            pltpu.SemaphoreType.DMA,
        ],
    )
    def k(row_h, deg_h, row_v, ones_v, z1, dg_sh, dsem):
        ci = lax.axis_index("c")
        t = lax.axis_index("s")
        wid = t * NC + ci

        def fill(r, _):
            ones_v[pl.ds(r * 16, 16)] = jnp.full((16,), 1.0, F32)
            return _
        lax.fori_loop(0, cg // 16, fill, None)

        def fillz(r, _):
            z1[pl.ds(r * 16, 16)] = jnp.zeros((16,), F32)
            return _
        lax.fori_loop(0, CGD // 16, fillz, None)

        def zloop(j, _):
            cz = t + j * NS

            @pl.when(cz < nz)
            def _():
                pltpu.sync_copy(z1, dg_sh.at[pl.ds(cz * CGD, CGD)])
            return _
        lax.fori_loop(0, (nz + NS - 1) // NS, zloop, None)
        plsc.subcore_barrier()

        pltpu.sync_copy(row_h.at[wid], row_v)

        def fire(c, _):
            pltpu.async_copy(ones_v, dg_sh.at[row_v.at[c]], dsem, add=True)
            return _
        lax.fori_loop(0, nch, fire, None)

        def drain(c, _):
            pltpu.make_async_copy(ones_v, dg_sh.at[row_v.at[c]],
                                  dsem).wait()
            return _
        lax.fori_loop(0, nch, drain, None)

        plsc.subcore_barrier()

        def dloop(j, _):
            cz = t + j * NS

            @pl.when(cz < nz)
            def _():
                pltpu.sync_copy(dg_sh.at[pl.ds(cz * CGD, CGD)], z1)
                pltpu.sync_copy(z1,
                                deg_h.at[pl.ds(ci * n_ent + cz * CGD, CGD)])
            return _
        lax.fori_loop(0, (nz + NS - 1) // NS, dloop, None)

    return k(row3d)


def _sc_normg(dinv128, row3d, n_edge):
    nch = row3d.shape[1]

    @functools.partial(
        pl.kernel,
        out_type=jax.ShapeDtypeStruct((n_edge, 16), F32),
        mesh=_mesh(),
        scratch_types=[
            pltpu.VMEM((nch, CG), jnp.int32),
            pltpu.VMEM((2, CG, 128), F32),
            pltpu.VMEM((2, CG, 16), F32),
            pltpu.SemaphoreType.DMA((2,)),
            pltpu.SemaphoreType.DMA((2,)),
        ],
    )
    def k(dinv_h, row_h, out_h, row_v, nb, nb16, gsem, wsem):
        ci = lax.axis_index("c")
        t = lax.axis_index("s")
        wid = t * NC + ci
        pltpu.sync_copy(row_h.at[wid], row_v)

        def fire(c, s):
            pltpu.async_copy(dinv_h.at[row_v.at[c]], nb.at[s], gsem.at[s])

        def wait_g(c, s):
            pltpu.make_async_copy(dinv_h.at[row_v.at[c]], nb.at[s],
                                  gsem.at[s]).wait()

        def wait_w(c, s):
            off = pl.ds((wid * nch + c) * CG, CG)
            pltpu.make_async_copy(nb16.at[s], out_h.at[off],
                                  wsem.at[s]).wait()

        fire(0, 0)

        def step(c, s):
            wait_g(c, s)
            o = 1 - s

            @pl.when(c + 1 < nch)
            def _():
                fire(c + 1, o)

            @pl.when(c >= 2)
            def _():
                wait_w(c - 2, s)

            def ext(r, _):
                nb16[s, r, :] = nb[s, r, pl.ds(0, 16)]
                return _
            lax.fori_loop(0, CG, ext, None)
            off = pl.ds((wid * nch + c) * CG, CG)
            pltpu.async_copy(nb16.at[s], out_h.at[off], wsem.at[s])

        def body(c, _):
            @pl.when(c % 2 == 0)
            def _():
                step(c, 0)

            @pl.when(c % 2 == 1)
            def _():
                step(c, 1)
            return _
        lax.fori_loop(0, nch, body, None)
        wait_w(nch - 1, (nch - 1) % 2)
        wait_w(nch - 2, (nch - 2) % 2)

    return k(dinv128, row3d)


def _sc_gather(ent, rel_e, xi, row3d, et3d, dst3d, n_edge):
    nch = row3d.shape[1]

    @functools.partial(
        pl.kernel,
        out_type=[
            jax.ShapeDtypeStruct((n_edge, 128), F32),
            jax.ShapeDtypeStruct((n_edge, 128), F32),
        ],
        mesh=_mesh(),
        scratch_types=[
            pltpu.VMEM((nch, CG), jnp.int32),
            pltpu.VMEM((nch, CG), jnp.int32),
            pltpu.VMEM((nch, CG), jnp.int32),
            pltpu.VMEM((2, CG, 128), F32),
            pltpu.VMEM((2, CG, 128), F32),
            pltpu.VMEM((2, CG, 128), F32),
            pltpu.SemaphoreType.DMA((2,)),
            pltpu.SemaphoreType.DMA((2,)),
        ],
    )
    def k(ent_h, rel_h, xi_h, row_h, et_h, dst_h, msg_h, xid_h,
          row_v, et_v, dst_v, xj_v, rl_v, xd_v, gsem, wsem):
        ci = lax.axis_index("c")
        t = lax.axis_index("s")
        wid = t * NC + ci
        pltpu.sync_copy(row_h.at[wid], row_v)
        pltpu.sync_copy(et_h.at[wid], et_v)
        pltpu.sync_copy(dst_h.at[wid], dst_v)

        def fire(c, s):
            pltpu.async_copy(ent_h.at[row_v.at[c]], xj_v.at[s], gsem.at[s])
            pltpu.async_copy(rel_h.at[et_v.at[c]], rl_v.at[s], gsem.at[s])
            pltpu.async_copy(xi_h.at[dst_v.at[c]], xd_v.at[s], gsem.at[s])

        def wait_g(c, s):
            pltpu.make_async_copy(ent_h.at[row_v.at[c]], xj_v.at[s],
                                  gsem.at[s]).wait()
            pltpu.make_async_copy(rel_h.at[et_v.at[c]], rl_v.at[s],
                                  gsem.at[s]).wait()
            pltpu.make_async_copy(xi_h.at[dst_v.at[c]], xd_v.at[s],
                                  gsem.at[s]).wait()

        def wait_w(c, s):
            off = pl.ds((wid * nch + c) * CG, CG)
            pltpu.make_async_copy(xj_v.at[s], msg_h.at[off],
                                  wsem.at[s]).wait()
            pltpu.make_async_copy(xd_v.at[s], xid_h.at[off],
                                  wsem.at[s]).wait()

        fire(0, 0)

        def step(c, s):
            wait_g(c, s)
            o = 1 - s

            @pl.when(c + 1 < nch)
            def _():
                @pl.when(c >= 1)
                def _():
                    wait_w(c - 1, o)
                fire(c + 1, o)

            @plsc.parallel_loop(0, CG, step=1, unroll=4)
            def mul(r):
                for kk in range(8):
                    d = pl.ds(kk * 16, 16)
                    xj_v[s, r, d] = xj_v[s, r, d] * rl_v[s, r, d]

            off = pl.ds((wid * nch + c) * CG, CG)
            pltpu.async_copy(xj_v.at[s], msg_h.at[off], wsem.at[s])
            pltpu.async_copy(xd_v.at[s], xid_h.at[off], wsem.at[s])

        def body(c, _):
            @pl.when(c % 2 == 0)
            def _():
                step(c, 0)

            @pl.when(c % 2 == 1)
            def _():
                step(c, 1)
            return _
        lax.fori_loop(0, nch, body, None)
        wait_w(nch - 1, (nch - 1) % 2)
        wait_w(nch - 2, (nch - 2) % 2)

    return k(ent, rel_e, xi, row3d, et3d, dst3d)


def _sc_scatter(msgs, gg, dst3d, n_ent, n_edge):
    nch = dst3d.shape[1]
    nz = n_ent // CGD

    @functools.partial(
        pl.kernel,
        out_type=[
            jax.ShapeDtypeStruct((NC, n_ent, 128), F32),
            jax.ShapeDtypeStruct((NC * n_ent,), F32),
        ],
        mesh=_mesh(),
        scratch_types=[
            pltpu.VMEM((nch, CG), jnp.int32),
            pltpu.VMEM((2, CG, 128), F32),
            pltpu.VMEM((2, CG), F32),
            pltpu.VMEM_SHARED((n_ent, 128), F32),
            pltpu.VMEM_SHARED((n_ent,), F32),
            pltpu.SemaphoreType.DMA((2,)),
            pltpu.SemaphoreType.DMA((2,)),
        ],
    )
    def k(msgs_h, gg_h, dst_h, u_out, dn_out,
          dst_v, mb, gb, u_sh, dn_sh, rsem, ssem):
        ci = lax.axis_index("c")
        t = lax.axis_index("s")
        wid = t * NC + ci

        def fill_z(r, _):
            for kk in range(8):
                mb[0, r, pl.ds(kk * 16, 16)] = jnp.zeros((16,), F32)
            return _
        lax.fori_loop(0, CGD, fill_z, None)

        def fill_z1(r, _):
            gb[0, pl.ds(r * 16, 16)] = jnp.zeros((16,), F32)
            return _
        lax.fori_loop(0, CGD // 16, fill_z1, None)

        def zloop(j, _):
            cz = t + j * NS

            @pl.when(cz < nz)
            def _():
                pltpu.sync_copy(mb.at[0, pl.ds(0, CGD)],
                                u_sh.at[pl.ds(cz * CGD, CGD)])
                pltpu.sync_copy(gb.at[0, pl.ds(0, CGD)],
                                dn_sh.at[pl.ds(cz * CGD, CGD)])
            return _
        lax.fori_loop(0, (nz + NS - 1) // NS, zloop, None)
        plsc.subcore_barrier()

        pltpu.sync_copy(dst_h.at[wid], dst_v)

        def fire_r(c, s):
            off = pl.ds((wid * nch + c) * CG, CG)
            pltpu.async_copy(msgs_h.at[off], mb.at[s], rsem.at[s])
            pltpu.async_copy(gg_h.at[off], gb.at[s], rsem.at[s])

        def wait_r(c, s):
            off = pl.ds((wid * nch + c) * CG, CG)
            pltpu.make_async_copy(msgs_h.at[off], mb.at[s],
                                  rsem.at[s]).wait()
            pltpu.make_async_copy(gg_h.at[off], gb.at[s],
                                  rsem.at[s]).wait()

        def wait_s(c, s):
            pltpu.make_async_copy(mb.at[s], u_sh.at[dst_v.at[c]],
                                  ssem.at[s]).wait()
            pltpu.make_async_copy(gb.at[s], dn_sh.at[dst_v.at[c]],
                                  ssem.at[s]).wait()

        fire_r(0, 0)

        def step(c, s):
            wait_r(c, s)
            o = 1 - s

            @pl.when(c + 1 < nch)
            def _():
                @pl.when(c >= 1)
                def _():
                    wait_s(c - 1, o)
                fire_r(c + 1, o)

            pltpu.async_copy(mb.at[s], u_sh.at[dst_v.at[c]], ssem.at[s],
                             add=True)
            pltpu.async_copy(gb.at[s], dn_sh.at[dst_v.at[c]], ssem.at[s],
                             add=True)

        def body(c, _):
            @pl.when(c % 2 == 0)
            def _():
                step(c, 0)

            @pl.when(c % 2 == 1)
            def _():
                step(c, 1)
            return _
        lax.fori_loop(0, nch, body, None)
        wait_s(nch - 1, (nch - 1) % 2)
        wait_s(nch - 2, (nch - 2) % 2)

        plsc.subcore_barrier()

        def dloop(j, _):
            cz = t + j * NS

            @pl.when(cz < nz)
            def _():
                sl = pl.ds(cz * CGD, CGD)
                pltpu.sync_copy(u_sh.at[sl], mb.at[0, pl.ds(0, CGD)])
                pltpu.sync_copy(mb.at[0, pl.ds(0, CGD)], u_out.at[ci, sl])
                pltpu.sync_copy(dn_sh.at[sl], gb.at[0, pl.ds(0, CGD)])
                pltpu.sync_copy(gb.at[0, pl.ds(0, CGD)],
                                dn_out.at[pl.ds(ci * n_ent + cz * CGD, CGD)])
            return _
        lax.fori_loop(0, (nz + NS - 1) // NS, dloop, None)

    return k(msgs, gg, dst3d)


def _mm(a, b):
    return jnp.dot(a, b, preferred_element_type=F32)


def _l2n(a):
    nrm = jnp.sqrt(jnp.sum(a * a, axis=-1, keepdims=True))
    return a / jnp.maximum(nrm, 1e-12)


def _bn_tanh(hp, ssum, ssq, g, b, n):
    mean = ssum / n
    var = ssq / n - mean * mean
    inv = 1.0 / jnp.sqrt(var + 1e-5)
    return jnp.tanh((hp - mean) * inv * g + b)


def _wspec():
    return pl.BlockSpec((128, 128), lambda i: (0, 0))


def _rspec():
    return pl.BlockSpec((RB, 128), lambda i: (i, 0))


def _tc_prep0(ec, ei, lw, rel_args):
    n = ec.shape[0]
    nbk = n // RB
    (rp1, w11, w21, rp2, w12, w22, rinfo, r11a, r12a, l3p, w13, w23) = rel_args
    (l11a, l11b, l12, wtop1, wtopc) = lw

    def body(ec_ref, ei_ref, l11a_r, l11b_r, l12_r, wt1_r, wtc_r,
             rp1_r, w11_r, w21_r, rp2_r, w12_r, w22_r,
             ri_r, r11a_r, r12a_r, l3p_r, w13_r, w23_r,
             e1_ref, xi1_ref, xic_ref, rel1_ref, rel2_ref,
             rel3a_ref, rel3b_ref):
        a = _l2n(ec_ref[...])
        e1 = _mm(_lrelu(_mm(a, l11a_r[...]) + _mm(ei_ref[...], l11b_r[...])),
                 l12_r[...])
        e1_ref[...] = e1
        xi1_ref[...] = _mm(e1, wt1_r[...])
        xic_ref[...] = _mm(ec_ref[...], wtc_r[...])

        @pl.when(pl.program_id(0) == 0)
        def _():
            rel1_ref[...] = _mm(_lrelu(_mm(rp1_r[...], w11_r[...])),
                                w21_r[...])
            rel2_ref[...] = _mm(_lrelu(_mm(rp2_r[...], w12_r[...])),
                                w22_r[...])
            ar = _mm(_lrelu(_mm(ri_r[...], r11a_r[...])), r12a_r[...])
            rel3a_ref[...] = _mm(_lrelu(_mm(ar, w13_r[...])), w23_r[...])
            rel3b_ref[...] = _mm(_lrelu(_mm(l3p_r[...], w13_r[...])),
                                 w23_r[...])

    rel_spec = lambda shape: pl.BlockSpec(shape, lambda i: (0, 0))
    return pl.pallas_call(
        body,
        grid=(nbk,),
        in_specs=[_rspec(), _rspec(),
                  _wspec(), _wspec(), _wspec(), _wspec(), _wspec(),
                  rel_spec((408, 128)), _wspec(), _wspec(),
                  rel_spec((408, 128)), _wspec(), _wspec(),
                  rel_spec((400, 128)), _wspec(), _wspec(),
                  rel_spec((8, 128)), _wspec(), _wspec()],
        out_specs=[_rspec(), _rspec(), _rspec(),
                   rel_spec((408, 128)), rel_spec((408, 128)),
                   rel_spec((400, 128)), rel_spec((8, 128))],
        out_shape=[jax.ShapeDtypeStruct((n, 128), F32),
                   jax.ShapeDtypeStruct((n, 128), F32),
                   jax.ShapeDtypeStruct((n, 128), F32),
                   jax.ShapeDtypeStruct((408, 128), F32),
                   jax.ShapeDtypeStruct((408, 128), F32),
                   jax.ShapeDtypeStruct((400, 128), F32),
                   jax.ShapeDtypeStruct((8, 128), F32)],
    )(ec, ei, l11a, l11b, l12, wtop1, wtopc,
      rp1, w11, w21, rp2, w12, w22, rinfo, r11a, r12a, l3p, w13, w23)


def _tc_deginv(d0, d1):
    n = d0.shape[0]
    nbk = n // RB

    def body(d0_ref, d1_ref, o_ref):
        d = d0_ref[...] + d1_ref[...]
        o_ref[...] = jnp.broadcast_to(
            jnp.where(d > 0, 1.0 / jnp.sqrt(d), 0.0), (RB, 128))

    return pl.pallas_call(
        body,
        grid=(nbk,),
        in_specs=[pl.BlockSpec((RB, 1), lambda i: (i, 0)),
                  pl.BlockSpec((RB, 1), lambda i: (i, 0))],
        out_specs=pl.BlockSpec((RB, 128), lambda i: (i, 0)),
        out_shape=jax.ShapeDtypeStruct((n, 128), F32),
    )(d0, d1)


def _tc_norm1(norm16, n_edge):
    neb = n_edge // EB

    def body(n_ref, o_ref):
        o_ref[...] = n_ref[:, 0:1].reshape(EB)

    return pl.pallas_call(
        body,
        grid=(neb,),
        in_specs=[pl.BlockSpec((EB, 16), lambda i: (i, 0))],
        out_specs=pl.BlockSpec((EB,), lambda i: (i,)),
        out_shape=jax.ShapeDtypeStruct((n_edge,), F32),
    )(norm16)


def _tc_score_scale(msg, xid, wb, aa, norm1, n_edge, n_real):
    neb = n_edge // EB

    def body(msg_ref, xid_ref, wb_ref, aa_ref, nrm_ref, msgs_ref, gg_ref):
        z = _lrelu(_mm(msg_ref[...], wb_ref[...]) + xid_ref[...])
        s = _mm(z, aa_ref[...])
        i = pl.program_id(0)
        eidx = lax.broadcasted_iota(jnp.int32, (EB, 1), 0) + i * EB
        g = jnp.where(eidx < n_real, jnp.exp(s), 0.0)
        gg_ref[...] = g.reshape(EB)
        gn = g * nrm_ref[...].reshape(EB, 1)
        msgs_ref[...] = msg_ref[...] * gn

    return pl.pallas_call(
        body,
        grid=(neb,),
        in_specs=[pl.BlockSpec((EB, 128), lambda i: (i, 0)),
                  pl.BlockSpec((EB, 128), lambda i: (i, 0)),
                  pl.BlockSpec((128, 128), lambda i: (0, 0)),
                  pl.BlockSpec((128, 1), lambda i: (0, 0)),
                  pl.BlockSpec((EB,), lambda i: (i,))],
        out_specs=[pl.BlockSpec((EB, 128), lambda i: (i, 0)),
                   pl.BlockSpec((EB,), lambda i: (i,))],
        out_shape=[jax.ShapeDtypeStruct((n_edge, 128), F32),
                   jax.ShapeDtypeStruct((n_edge,), F32)],
    )(msg, xid, wb, aa, norm1)


def _tc_comb_a(u0, u1, d0, d1, ent, loopv, gcn):
    n = ent.shape[0]
    nbk = n // RB

    def body(u0_ref, u1_ref, d0_ref, d1_ref, ent_ref, lv_ref, g_ref,
             hp_ref, ssum_ref, ssq_ref):
        den = d0_ref[...] + d1_ref[...] + 1e-16
        pre = (u0_ref[...] + u1_ref[...]) / den + ent_ref[...] * lv_ref[...]
        hp = _mm(pre, g_ref[...]) * 0.5
        hp_ref[...] = hp
        cs = jnp.sum(hp, axis=0, keepdims=True)
        cq = jnp.sum(hp * hp, axis=0, keepdims=True)
        i = pl.program_id(0)

        @pl.when(i == 0)
        def _():
            ssum_ref[...] = cs
            ssq_ref[...] = cq

        @pl.when(i > 0)
        def _():
            ssum_ref[...] = ssum_ref[...] + cs
            ssq_ref[...] = ssq_ref[...] + cq

    vspec = pl.BlockSpec((1, 128), lambda i: (0, 0))
    return pl.pallas_call(
        body,
        grid=(nbk,),
        in_specs=[_rspec(), _rspec(),
                  pl.BlockSpec((RB, 1), lambda i: (i, 0)),
                  pl.BlockSpec((RB, 1), lambda i: (i, 0)),
                  _rspec(), vspec, _wspec()],
        out_specs=[_rspec(), vspec, vspec],
        out_shape=[jax.ShapeDtypeStruct((n, 128), F32),
                   jax.ShapeDtypeStruct((1, 128), F32),
                   jax.ShapeDtypeStruct((1, 128), F32)],
    )(u0, u1, d0, d1, ent, loopv, gcn)


def _tc_e2m(hpa, suma, sqa, bga, bba, hpc, sumc, sqc, bgc, bbc,
            l21a, l21b, l22, wtop2):
    n = hpa.shape[0]
    nbk = n // RB

    def body(hpa_r, sa_r, qa_r, ga_r, ba_r, hpc_r, sc_r, qc_r, gc_r, bc_r,
             l21a_r, l21b_r, l22_r, wt2_r, a1_ref, e2_ref, xi2_ref):
        a1 = _bn_tanh(hpa_r[...], sa_r[...], qa_r[...], ga_r[...],
                      ba_r[...], n)
        c1 = _bn_tanh(hpc_r[...], sc_r[...], qc_r[...], gc_r[...],
                      bc_r[...], n)
        a1_ref[...] = a1
        e2 = _mm(_lrelu(_mm(_l2n(c1), l21a_r[...]) + _mm(a1, l21b_r[...])),
                 l22_r[...])
        e2_ref[...] = e2
        xi2_ref[...] = _mm(e2, wt2_r[...])

    vspec = pl.BlockSpec((1, 128), lambda i: (0, 0))
    return pl.pallas_call(
        body,
        grid=(nbk,),
        in_specs=[_rspec(), vspec, vspec, vspec, vspec,
                  _rspec(), vspec, vspec, vspec, vspec,
                  _wspec(), _wspec(), _wspec(), _wspec()],
        out_specs=[_rspec(), _rspec(), _rspec()],
        out_shape=[jax.ShapeDtypeStruct((n, 128), F32),
                   jax.ShapeDtypeStruct((n, 128), F32),
                   jax.ShapeDtypeStruct((n, 128), F32)],
    )(hpa, suma, sqa, bga, bba, hpc, sumc, sqc, bgc, bbc,
      l21a, l21b, l22, wtop2)


def _tc_final(hp2, sum2, sq2, bg2, bb2, e1, a1, wa, wb, wc):
    n = hp2.shape[0]
    nbk = n // RB

    def body(hp_r, s_r, q_r, g_r, b_r, e1_r, a1_r, wa_r, wb_r, wc_r, o_ref):
        a2 = _bn_tanh(hp_r[...], s_r[...], q_r[...], g_r[...], b_r[...], n)
        o_ref[...] = (_mm(e1_r[...], wa_r[...]) + _mm(a1_r[...], wb_r[...])
                      + _mm(a2, wc_r[...]))

    vspec = pl.BlockSpec((1, 128), lambda i: (0, 0))
    return pl.pallas_call(
        body,
        grid=(nbk,),
        in_specs=[_rspec(), vspec, vspec, vspec, vspec,
                  _rspec(), _rspec(), _wspec(), _wspec(), _wspec()],
        out_specs=_rspec(),
        out_shape=jax.ShapeDtypeStruct((n, 128), F32),
    )(hp2, sum2, sq2, bg2, bb2, e1, a1, wa, wb, wc)


def _edge_phase(cp, ent, xi, rel_e, loopv, row3d, et3d, dst3d, norm1,
                n_ent, n_edge, n_real):
    msg, xid = _sc_gather(ent, rel_e, xi, row3d, et3d, dst3d, n_edge)
    msgs, gg = _tc_score_scale(msg, xid, cp['w_att'][128:], cp['a_att'],
                               norm1, n_edge, n_real)
    u2, dn2 = _sc_scatter(msgs, gg, dst3d, n_ent, n_edge)
    return _tc_comb_a(u2[0], u2[1],
                      dn2[:n_ent].reshape(n_ent, 1),
                      dn2[n_ent:].reshape(n_ent, 1),
                      ent, loopv, cp['gcn_w'])


def kernel(params, edge_index, edge_type):
    p = params
    cpa = p['conv1_align']
    cpc = p['conv1_completion']
    cp2 = p['conv2_align']
    n_ent = p['ent_completion_att'].shape[0]
    n_real = edge_type.shape[0]
    n_edge = ((n_real + NW * CG * 16 - 1) // (NW * CG * 16)) * (NW * CG * 16)
    nch = n_edge // (NW * CG)
    npad = n_edge - n_real

    row = edge_index[0]
    dst = edge_index[1]
    zi = jnp.zeros((npad,), jnp.int32)
    rowp = jnp.concatenate([row, zi])
    row3d = rowp.reshape(NW, nch, CG)
    et3d = jnp.concatenate([edge_type, zi]).reshape(NW, nch, CG)
    dst3d = jnp.concatenate([dst, zi]).reshape(NW, nch, CG)
    nchd = n_real // (NW * CGD)
    row3dd = row.reshape(NW, nchd, CGD)

    deg2 = _sc_deg(row3dd, n_ent)

    zpad = jnp.zeros((7, 128), F32)
    rp1 = jnp.concatenate([p['rel_info_att'], cpa['loop_rel'], zpad], axis=0)
    rp2 = jnp.concatenate([p['rel_completion_att'], cpc['loop_rel'], zpad],
                          axis=0)
    l3p = jnp.concatenate([cp2['loop_rel'], zpad], axis=0)
    e1, xi1, xic, rel1, rel2, rel3a, rel3b = _tc_prep0(
        p['ent_completion_att'], p['ent_info_att'],
        (p['align_linear1_1'][:128], p['align_linear1_1'][128:],
         p['align_linear1_2'], cpa['w_att'][:128], cpc['w_att'][:128]),
        (rp1, cpa['w1'], cpa['w2'], rp2, cpc['w1'], cpc['w2'],
         p['rel_info_att'], p['rel_linear11_align'], p['rel_linear12_align'],
         l3p, cp2['w1'], cp2['w2']))

    dinv128 = _tc_deginv(deg2[:n_ent].reshape(n_ent, 1),
                         deg2[n_ent:].reshape(n_ent, 1))
    norm16 = _sc_normg(dinv128, row3d, n_edge)
    norm1 = _tc_norm1(norm16, n_edge)

    hpa, suma, sqa = _edge_phase(cpa, e1, xi1, rel1, rel1[400:401],
                                 row3d, et3d, dst3d, norm1, n_ent, n_edge,
                                 n_real)
    hpc, sumc, sqc = _edge_phase(cpc, p['ent_completion_att'], xic, rel2,
                                 rel2[400:401], row3d, et3d, dst3d, norm1,
                                 n_ent, n_edge, n_real)

    a1, e2, xi2 = _tc_e2m(hpa, suma, sqa,
                          cpa['bn_g'].reshape(1, 128),
                          cpa['bn_b'].reshape(1, 128),
                          hpc, sumc, sqc,
                          cpc['bn_g'].reshape(1, 128),
                          cpc['bn_b'].reshape(1, 128),
                          p['align_linear2_1'][:128],
                          p['align_linear2_1'][128:],
                          p['align_linear2_2'], cp2['w_att'][:128])

    rel3 = jnp.concatenate([rel3a, rel3b], axis=0)
    hp2, sum2, sq2 = _edge_phase(cp2, e2, xi2, rel3, rel3b[0:1],
                                 row3d, et3d, dst3d, norm1, n_ent, n_edge,
                                 n_real)

    w = p['all_linear_comp']
    return _tc_final(hp2, sum2, sq2,
                     cp2['bn_g'].reshape(1, 128),
                     cp2['bn_b'].reshape(1, 128),
                     e1, a1, w[:128], w[128:256], w[256:384])


# speedup vs baseline: 1.0972x; 1.0972x over previous
import functools

import jax
import jax.numpy as jnp
from jax import lax
from jax.experimental import pallas as pl
from jax.experimental.pallas import tpu as pltpu
from jax.experimental.pallas import tpu_sc as plsc

SLOPE = 0.2
NC = 2
NS = 16
NW = NC * NS
CG = 80
CGD = 80
RB = 1000
EB = 2048
F32 = jnp.float32


def _lrelu(x):
    return jnp.where(x >= 0, x, SLOPE * x)


def _mesh():
    return plsc.VectorSubcoreMesh(core_axis_name="c", subcore_axis_name="s",
                                  num_cores=NC, num_subcores=NS)


def _sc_deg(row3d, n_ent):
    nch = row3d.shape[1]
    cg = row3d.shape[2]
    nz = n_ent // CGD

    @functools.partial(
        pl.kernel,
        out_type=jax.ShapeDtypeStruct((NC * n_ent,), F32),
        mesh=_mesh(),
        scratch_types=[
            pltpu.VMEM((nch, cg), jnp.int32),
            pltpu.VMEM((cg,), F32),
            pltpu.VMEM((CGD,), F32),
            pltpu.VMEM_SHARED((n_ent,), F32),
            pltpu.SemaphoreType.DMA,
        ],
    )
    def k(row_h, deg_h, row_v, ones_v, z1, dg_sh, dsem):
        ci = lax.axis_index("c")
        t = lax.axis_index("s")
        wid = t * NC + ci

        def fill(r, _):
            ones_v[pl.ds(r * 16, 16)] = jnp.full((16,), 1.0, F32)
            return _
        lax.fori_loop(0, cg // 16, fill, None)

        def fillz(r, _):
            z1[pl.ds(r * 16, 16)] = jnp.zeros((16,), F32)
            return _
        lax.fori_loop(0, CGD // 16, fillz, None)

        def zloop(j, _):
            cz = t + j * NS

            @pl.when(cz < nz)
            def _():
                pltpu.sync_copy(z1, dg_sh.at[pl.ds(cz * CGD, CGD)])
            return _
        lax.fori_loop(0, (nz + NS - 1) // NS, zloop, None)
        plsc.subcore_barrier()

        pltpu.sync_copy(row_h.at[wid], row_v)

        def fire(c, _):
            pltpu.async_copy(ones_v, dg_sh.at[row_v.at[c]], dsem, add=True)
            return _
        lax.fori_loop(0, nch, fire, None)

        def drain(c, _):
            pltpu.make_async_copy(ones_v, dg_sh.at[row_v.at[c]],
                                  dsem).wait()
            return _
        lax.fori_loop(0, nch, drain, None)

        plsc.subcore_barrier()

        def dloop(j, _):
            cz = t + j * NS

            @pl.when(cz < nz)
            def _():
                pltpu.sync_copy(dg_sh.at[pl.ds(cz * CGD, CGD)], z1)
                pltpu.sync_copy(z1,
                                deg_h.at[pl.ds(ci * n_ent + cz * CGD, CGD)])
            return _
        lax.fori_loop(0, (nz + NS - 1) // NS, dloop, None)

    return k(row3d)


def _sc_normg(dinv128, row3d, n_edge):
    nch = row3d.shape[1]

    @functools.partial(
        pl.kernel,
        out_type=jax.ShapeDtypeStruct((n_edge, 16), F32),
        mesh=_mesh(),
        scratch_types=[
            pltpu.VMEM((nch, CG), jnp.int32),
            pltpu.VMEM((2, CG, 128), F32),
            pltpu.VMEM((2, CG, 16), F32),
            pltpu.SemaphoreType.DMA((2,)),
            pltpu.SemaphoreType.DMA((2,)),
        ],
    )
    def k(dinv_h, row_h, out_h, row_v, nb, nb16, gsem, wsem):
        ci = lax.axis_index("c")
        t = lax.axis_index("s")
        wid = t * NC + ci
        pltpu.sync_copy(row_h.at[wid], row_v)

        def fire(c, s):
            pltpu.async_copy(dinv_h.at[row_v.at[c]], nb.at[s], gsem.at[s])

        def wait_g(c, s):
            pltpu.make_async_copy(dinv_h.at[row_v.at[c]], nb.at[s],
                                  gsem.at[s]).wait()

        def wait_w(c, s):
            off = pl.ds((wid * nch + c) * CG, CG)
            pltpu.make_async_copy(nb16.at[s], out_h.at[off],
                                  wsem.at[s]).wait()

        fire(0, 0)

        def step(c, s):
            wait_g(c, s)
            o = 1 - s

            @pl.when(c + 1 < nch)
            def _():
                fire(c + 1, o)

            @pl.when(c >= 2)
            def _():
                wait_w(c - 2, s)

            def ext(r, _):
                nb16[s, r, :] = nb[s, r, pl.ds(0, 16)]
                return _
            lax.fori_loop(0, CG, ext, None)
            off = pl.ds((wid * nch + c) * CG, CG)
            pltpu.async_copy(nb16.at[s], out_h.at[off], wsem.at[s])

        def body(c, _):
            @pl.when(c % 2 == 0)
            def _():
                step(c, 0)

            @pl.when(c % 2 == 1)
            def _():
                step(c, 1)
            return _
        lax.fori_loop(0, nch, body, None)
        wait_w(nch - 1, (nch - 1) % 2)
        wait_w(nch - 2, (nch - 2) % 2)

    return k(dinv128, row3d)


def _sc_gather(ent, rel_e, xi, row3d, et3d, dst3d, n_edge):
    nch = row3d.shape[1]

    @functools.partial(
        pl.kernel,
        out_type=[
            jax.ShapeDtypeStruct((n_edge, 128), F32),
            jax.ShapeDtypeStruct((n_edge, 128), F32),
        ],
        mesh=_mesh(),
        scratch_types=[
            pltpu.VMEM((nch, CG), jnp.int32),
            pltpu.VMEM((nch, CG), jnp.int32),
            pltpu.VMEM((nch, CG), jnp.int32),
            pltpu.VMEM((2, CG, 128), F32),
            pltpu.VMEM((2, CG, 128), F32),
            pltpu.VMEM((2, CG, 128), F32),
            pltpu.VMEM((2, CG, 128), F32),
            pltpu.SemaphoreType.DMA((2,)),
            pltpu.SemaphoreType.DMA((2,)),
        ],
    )
    def k(ent_h, rel_h, xi_h, row_h, et_h, dst_h, msg_h, xid_h,
          row_v, et_v, dst_v, xj_v, rl_v, xd_v, mo_v, gsem, wsem):
        ci = lax.axis_index("c")
        t = lax.axis_index("s")
        wid = t * NC + ci
        pltpu.sync_copy(row_h.at[wid], row_v)
        pltpu.sync_copy(et_h.at[wid], et_v)
        pltpu.sync_copy(dst_h.at[wid], dst_v)

        def fire(c, s):
            pltpu.async_copy(ent_h.at[row_v.at[c]], xj_v.at[s], gsem.at[s])
            pltpu.async_copy(rel_h.at[et_v.at[c]], rl_v.at[s], gsem.at[s])
            pltpu.async_copy(xi_h.at[dst_v.at[c]], xd_v.at[s], gsem.at[s])

        def wait_g(c, s):
            pltpu.make_async_copy(ent_h.at[row_v.at[c]], xj_v.at[s],
                                  gsem.at[s]).wait()
            pltpu.make_async_copy(rel_h.at[et_v.at[c]], rl_v.at[s],
                                  gsem.at[s]).wait()
            pltpu.make_async_copy(xi_h.at[dst_v.at[c]], xd_v.at[s],
                                  gsem.at[s]).wait()

        def wait_w(c, s):
            off = pl.ds((wid * nch + c) * CG, CG)
            pltpu.make_async_copy(mo_v.at[s], msg_h.at[off],
                                  wsem.at[s]).wait()
            pltpu.make_async_copy(xd_v.at[s], xid_h.at[off],
                                  wsem.at[s]).wait()

        fire(0, 0)

        def step(c, s):
            wait_g(c, s)
            o = 1 - s

            @pl.when(c + 1 < nch)
            def _():
                @pl.when(c >= 1)
                def _():
                    wait_w(c - 1, o)
                fire(c + 1, o)

            @plsc.parallel_loop(0, CG, step=1, unroll=4)
            def mul(r):
                for kk in range(8):
                    d = pl.ds(kk * 16, 16)
                    mo_v[s, r, d] = xj_v[s, r, d] * rl_v[s, r, d]

            off = pl.ds((wid * nch + c) * CG, CG)
            pltpu.async_copy(mo_v.at[s], msg_h.at[off], wsem.at[s])
            pltpu.async_copy(xd_v.at[s], xid_h.at[off], wsem.at[s])

        def body(c, _):
            @pl.when(c % 2 == 0)
            def _():
                step(c, 0)

            @pl.when(c % 2 == 1)
            def _():
                step(c, 1)
            return _
        lax.fori_loop(0, nch, body, None)
        wait_w(nch - 1, (nch - 1) % 2)
        wait_w(nch - 2, (nch - 2) % 2)

    return k(ent, rel_e, xi, row3d, et3d, dst3d)


def _sc_scatter(msgs, gg, dst3d, n_ent, n_edge):
    nch = dst3d.shape[1]
    nz = n_ent // CGD

    @functools.partial(
        pl.kernel,
        out_type=[
            jax.ShapeDtypeStruct((NC, n_ent, 128), F32),
            jax.ShapeDtypeStruct((NC * n_ent,), F32),
        ],
        mesh=_mesh(),
        scratch_types=[
            pltpu.VMEM((nch, CG), jnp.int32),
            pltpu.VMEM((2, CG, 128), F32),
            pltpu.VMEM((2, CG), F32),
            pltpu.VMEM_SHARED((n_ent, 128), F32),
            pltpu.VMEM_SHARED((n_ent,), F32),
            pltpu.SemaphoreType.DMA((2,)),
            pltpu.SemaphoreType.DMA((2,)),
        ],
    )
    def k(msgs_h, gg_h, dst_h, u_out, dn_out,
          dst_v, mb, gb, u_sh, dn_sh, rsem, ssem):
        ci = lax.axis_index("c")
        t = lax.axis_index("s")
        wid = t * NC + ci

        def fill_z(r, _):
            for kk in range(8):
                mb[0, r, pl.ds(kk * 16, 16)] = jnp.zeros((16,), F32)
            return _
        lax.fori_loop(0, CGD, fill_z, None)

        def fill_z1(r, _):
            gb[0, pl.ds(r * 16, 16)] = jnp.zeros((16,), F32)
            return _
        lax.fori_loop(0, CGD // 16, fill_z1, None)

        def zloop(j, _):
            cz = t + j * NS

            @pl.when(cz < nz)
            def _():
                pltpu.sync_copy(mb.at[0, pl.ds(0, CGD)],
                                u_sh.at[pl.ds(cz * CGD, CGD)])
                pltpu.sync_copy(gb.at[0, pl.ds(0, CGD)],
                                dn_sh.at[pl.ds(cz * CGD, CGD)])
            return _
        lax.fori_loop(0, (nz + NS - 1) // NS, zloop, None)
        plsc.subcore_barrier()

        pltpu.sync_copy(dst_h.at[wid], dst_v)

        def fire_r(c, s):
            off = pl.ds((wid * nch + c) * CG, CG)
            pltpu.async_copy(msgs_h.at[off], mb.at[s], rsem.at[s])
            pltpu.async_copy(gg_h.at[off], gb.at[s], rsem.at[s])

        def wait_r(c, s):
            off = pl.ds((wid * nch + c) * CG, CG)
            pltpu.make_async_copy(msgs_h.at[off], mb.at[s],
                                  rsem.at[s]).wait()
            pltpu.make_async_copy(gg_h.at[off], gb.at[s],
                                  rsem.at[s]).wait()

        def wait_s(c, s):
            pltpu.make_async_copy(mb.at[s], u_sh.at[dst_v.at[c]],
                                  ssem.at[s]).wait()
            pltpu.make_async_copy(gb.at[s], dn_sh.at[dst_v.at[c]],
                                  ssem.at[s]).wait()

        fire_r(0, 0)

        def step(c, s):
            wait_r(c, s)
            o = 1 - s

            @pl.when(c + 1 < nch)
            def _():
                @pl.when(c >= 1)
                def _():
                    wait_s(c - 1, o)
                fire_r(c + 1, o)

            pltpu.async_copy(mb.at[s], u_sh.at[dst_v.at[c]], ssem.at[s],
                             add=True)
            pltpu.async_copy(gb.at[s], dn_sh.at[dst_v.at[c]], ssem.at[s],
                             add=True)

        def body(c, _):
            @pl.when(c % 2 == 0)
            def _():
                step(c, 0)

            @pl.when(c % 2 == 1)
            def _():
                step(c, 1)
            return _
        lax.fori_loop(0, nch, body, None)
        wait_s(nch - 1, (nch - 1) % 2)
        wait_s(nch - 2, (nch - 2) % 2)

        plsc.subcore_barrier()

        def dloop(j, _):
            cz = t + j * NS

            @pl.when(cz < nz)
            def _():
                sl = pl.ds(cz * CGD, CGD)
                pltpu.sync_copy(u_sh.at[sl], mb.at[0, pl.ds(0, CGD)])
                pltpu.sync_copy(mb.at[0, pl.ds(0, CGD)], u_out.at[ci, sl])
                pltpu.sync_copy(dn_sh.at[sl], gb.at[0, pl.ds(0, CGD)])
                pltpu.sync_copy(gb.at[0, pl.ds(0, CGD)],
                                dn_out.at[pl.ds(ci * n_ent + cz * CGD, CGD)])
            return _
        lax.fori_loop(0, (nz + NS - 1) // NS, dloop, None)

    return k(msgs, gg, dst3d)


def _mm(a, b):
    return jnp.dot(a, b, preferred_element_type=F32)


def _l2n(a):
    nrm = jnp.sqrt(jnp.sum(a * a, axis=-1, keepdims=True))
    return a / jnp.maximum(nrm, 1e-12)


def _bn_tanh(hp, ssum, ssq, g, b, n):
    mean = ssum / n
    var = ssq / n - mean * mean
    inv = 1.0 / jnp.sqrt(var + 1e-5)
    return jnp.tanh((hp - mean) * inv * g + b)


def _wspec():
    return pl.BlockSpec((128, 128), lambda i: (0, 0))


def _rspec():
    return pl.BlockSpec((RB, 128), lambda i: (i, 0))


def _tc_prep0(ec, ei, lw, rel_args):
    n = ec.shape[0]
    nbk = n // RB
    (rp1, w11, w21, rp2, w12, w22, rinfo, r11a, r12a, l3p, w13, w23) = rel_args
    (l11a, l11b, l12, wtop1, wtopc) = lw

    def body(ec_ref, ei_ref, l11a_r, l11b_r, l12_r, wt1_r, wtc_r,
             rp1_r, w11_r, w21_r, rp2_r, w12_r, w22_r,
             ri_r, r11a_r, r12a_r, l3p_r, w13_r, w23_r,
             e1_ref, xi1_ref, xic_ref, rel1_ref, rel2_ref,
             rel3a_ref, rel3b_ref):
        a = _l2n(ec_ref[...])
        e1 = _mm(_lrelu(_mm(a, l11a_r[...]) + _mm(ei_ref[...], l11b_r[...])),
                 l12_r[...])
        e1_ref[...] = e1
        xi1_ref[...] = _mm(e1, wt1_r[...])
        xic_ref[...] = _mm(ec_ref[...], wtc_r[...])

        @pl.when(pl.program_id(0) == 0)
        def _():
            rel1_ref[...] = _mm(_lrelu(_mm(rp1_r[...], w11_r[...])),
                                w21_r[...])
            rel2_ref[...] = _mm(_lrelu(_mm(rp2_r[...], w12_r[...])),
                                w22_r[...])
            ar = _mm(_lrelu(_mm(ri_r[...], r11a_r[...])), r12a_r[...])
            rel3a_ref[...] = _mm(_lrelu(_mm(ar, w13_r[...])), w23_r[...])
            rel3b_ref[...] = _mm(_lrelu(_mm(l3p_r[...], w13_r[...])),
                                 w23_r[...])

    rel_spec = lambda shape: pl.BlockSpec(shape, lambda i: (0, 0))
    return pl.pallas_call(
        body,
        grid=(nbk,),
        in_specs=[_rspec(), _rspec(),
                  _wspec(), _wspec(), _wspec(), _wspec(), _wspec(),
                  rel_spec((408, 128)), _wspec(), _wspec(),
                  rel_spec((408, 128)), _wspec(), _wspec(),
                  rel_spec((400, 128)), _wspec(), _wspec(),
                  rel_spec((8, 128)), _wspec(), _wspec()],
        out_specs=[_rspec(), _rspec(), _rspec(),
                   rel_spec((408, 128)), rel_spec((408, 128)),
                   rel_spec((400, 128)), rel_spec((8, 128))],
        out_shape=[jax.ShapeDtypeStruct((n, 128), F32),
                   jax.ShapeDtypeStruct((n, 128), F32),
                   jax.ShapeDtypeStruct((n, 128), F32),
                   jax.ShapeDtypeStruct((408, 128), F32),
                   jax.ShapeDtypeStruct((408, 128), F32),
                   jax.ShapeDtypeStruct((400, 128), F32),
                   jax.ShapeDtypeStruct((8, 128), F32)],
    )(ec, ei, l11a, l11b, l12, wtop1, wtopc,
      rp1, w11, w21, rp2, w12, w22, rinfo, r11a, r12a, l3p, w13, w23)


def _tc_deginv(d0, d1):
    n = d0.shape[0]
    nbk = n // RB

    def body(d0_ref, d1_ref, o_ref):
        d = d0_ref[...] + d1_ref[...]
        o_ref[...] = jnp.broadcast_to(
            jnp.where(d > 0, 1.0 / jnp.sqrt(d), 0.0), (RB, 128))

    return pl.pallas_call(
        body,
        grid=(nbk,),
        in_specs=[pl.BlockSpec((RB, 1), lambda i: (i, 0)),
                  pl.BlockSpec((RB, 1), lambda i: (i, 0))],
        out_specs=pl.BlockSpec((RB, 128), lambda i: (i, 0)),
        out_shape=jax.ShapeDtypeStruct((n, 128), F32),
    )(d0, d1)


def _tc_norm1(norm16, n_edge):
    neb = n_edge // EB

    def body(n_ref, o_ref):
        o_ref[...] = n_ref[:, 0:1].reshape(EB)

    return pl.pallas_call(
        body,
        grid=(neb,),
        in_specs=[pl.BlockSpec((EB, 16), lambda i: (i, 0))],
        out_specs=pl.BlockSpec((EB,), lambda i: (i,)),
        out_shape=jax.ShapeDtypeStruct((n_edge,), F32),
    )(norm16)


def _tc_score_scale(msg, xid, wb, aa, norm1, n_edge, n_real):
    neb = n_edge // EB

    def body(msg_ref, xid_ref, wb_ref, aa_ref, nrm_ref, msgs_ref, gg_ref):
        z = _lrelu(_mm(msg_ref[...], wb_ref[...]) + xid_ref[...])
        s = _mm(z, aa_ref[...])
        i = pl.program_id(0)
        eidx = lax.broadcasted_iota(jnp.int32, (EB, 1), 0) + i * EB
        g = jnp.where(eidx < n_real, jnp.exp(s), 0.0)
        gg_ref[...] = g.reshape(EB)
        gn = g * nrm_ref[...].reshape(EB, 1)
        msgs_ref[...] = msg_ref[...] * gn

    return pl.pallas_call(
        body,
        grid=(neb,),
        in_specs=[pl.BlockSpec((EB, 128), lambda i: (i, 0)),
                  pl.BlockSpec((EB, 128), lambda i: (i, 0)),
                  pl.BlockSpec((128, 128), lambda i: (0, 0)),
                  pl.BlockSpec((128, 1), lambda i: (0, 0)),
                  pl.BlockSpec((EB,), lambda i: (i,))],
        out_specs=[pl.BlockSpec((EB, 128), lambda i: (i, 0)),
                   pl.BlockSpec((EB,), lambda i: (i,))],
        out_shape=[jax.ShapeDtypeStruct((n_edge, 128), F32),
                   jax.ShapeDtypeStruct((n_edge,), F32)],
    )(msg, xid, wb, aa, norm1)


def _tc_comb_a(u0, u1, d0, d1, ent, loopv, gcn):
    n = ent.shape[0]
    nbk = n // RB

    def body(u0_ref, u1_ref, d0_ref, d1_ref, ent_ref, lv_ref, g_ref,
             hp_ref, ssum_ref, ssq_ref):
        den = d0_ref[...] + d1_ref[...] + 1e-16
        pre = (u0_ref[...] + u1_ref[...]) / den + ent_ref[...] * lv_ref[...]
        hp = _mm(pre, g_ref[...]) * 0.5
        hp_ref[...] = hp
        cs = jnp.sum(hp, axis=0, keepdims=True)
        cq = jnp.sum(hp * hp, axis=0, keepdims=True)
        i = pl.program_id(0)

        @pl.when(i == 0)
        def _():
            ssum_ref[...] = cs
            ssq_ref[...] = cq

        @pl.when(i > 0)
        def _():
            ssum_ref[...] = ssum_ref[...] + cs
            ssq_ref[...] = ssq_ref[...] + cq

    vspec = pl.BlockSpec((1, 128), lambda i: (0, 0))
    return pl.pallas_call(
        body,
        grid=(nbk,),
        in_specs=[_rspec(), _rspec(),
                  pl.BlockSpec((RB, 1), lambda i: (i, 0)),
                  pl.BlockSpec((RB, 1), lambda i: (i, 0)),
                  _rspec(), vspec, _wspec()],
        out_specs=[_rspec(), vspec, vspec],
        out_shape=[jax.ShapeDtypeStruct((n, 128), F32),
                   jax.ShapeDtypeStruct((1, 128), F32),
                   jax.ShapeDtypeStruct((1, 128), F32)],
    )(u0, u1, d0, d1, ent, loopv, gcn)


def _tc_e2m(hpa, suma, sqa, bga, bba, hpc, sumc, sqc, bgc, bbc,
            l21a, l21b, l22, wtop2):
    n = hpa.shape[0]
    nbk = n // RB

    def body(hpa_r, sa_r, qa_r, ga_r, ba_r, hpc_r, sc_r, qc_r, gc_r, bc_r,
             l21a_r, l21b_r, l22_r, wt2_r, a1_ref, e2_ref, xi2_ref):
        a1 = _bn_tanh(hpa_r[...], sa_r[...], qa_r[...], ga_r[...],
                      ba_r[...], n)
        c1 = _bn_tanh(hpc_r[...], sc_r[...], qc_r[...], gc_r[...],
                      bc_r[...], n)
        a1_ref[...] = a1
        e2 = _mm(_lrelu(_mm(_l2n(c1), l21a_r[...]) + _mm(a1, l21b_r[...])),
                 l22_r[...])
        e2_ref[...] = e2
        xi2_ref[...] = _mm(e2, wt2_r[...])

    vspec = pl.BlockSpec((1, 128), lambda i: (0, 0))
    return pl.pallas_call(
        body,
        grid=(nbk,),
        in_specs=[_rspec(), vspec, vspec, vspec, vspec,
                  _rspec(), vspec, vspec, vspec, vspec,
                  _wspec(), _wspec(), _wspec(), _wspec()],
        out_specs=[_rspec(), _rspec(), _rspec()],
        out_shape=[jax.ShapeDtypeStruct((n, 128), F32),
                   jax.ShapeDtypeStruct((n, 128), F32),
                   jax.ShapeDtypeStruct((n, 128), F32)],
    )(hpa, suma, sqa, bga, bba, hpc, sumc, sqc, bgc, bbc,
      l21a, l21b, l22, wtop2)


def _tc_final(hp2, sum2, sq2, bg2, bb2, e1, a1, wa, wb, wc):
    n = hp2.shape[0]
    nbk = n // RB

    def body(hp_r, s_r, q_r, g_r, b_r, e1_r, a1_r, wa_r, wb_r, wc_r, o_ref):
        a2 = _bn_tanh(hp_r[...], s_r[...], q_r[...], g_r[...], b_r[...], n)
        o_ref[...] = (_mm(e1_r[...], wa_r[...]) + _mm(a1_r[...], wb_r[...])
                      + _mm(a2, wc_r[...]))

    vspec = pl.BlockSpec((1, 128), lambda i: (0, 0))
    return pl.pallas_call(
        body,
        grid=(nbk,),
        in_specs=[_rspec(), vspec, vspec, vspec, vspec,
                  _rspec(), _rspec(), _wspec(), _wspec(), _wspec()],
        out_specs=_rspec(),
        out_shape=jax.ShapeDtypeStruct((n, 128), F32),
    )(hp2, sum2, sq2, bg2, bb2, e1, a1, wa, wb, wc)


def _edge_phase(cp, ent, xi, rel_e, loopv, row3d, et3d, dst3d, norm1,
                n_ent, n_edge, n_real):
    msg, xid = _sc_gather(ent, rel_e, xi, row3d, et3d, dst3d, n_edge)
    msgs, gg = _tc_score_scale(msg, xid, cp['w_att'][128:], cp['a_att'],
                               norm1, n_edge, n_real)
    u2, dn2 = _sc_scatter(msgs, gg, dst3d, n_ent, n_edge)
    return _tc_comb_a(u2[0], u2[1],
                      dn2[:n_ent].reshape(n_ent, 1),
                      dn2[n_ent:].reshape(n_ent, 1),
                      ent, loopv, cp['gcn_w'])


def kernel(params, edge_index, edge_type):
    p = params
    cpa = p['conv1_align']
    cpc = p['conv1_completion']
    cp2 = p['conv2_align']
    n_ent = p['ent_completion_att'].shape[0]
    n_real = edge_type.shape[0]
    n_edge = ((n_real + NW * CG * 16 - 1) // (NW * CG * 16)) * (NW * CG * 16)
    nch = n_edge // (NW * CG)
    npad = n_edge - n_real

    row = edge_index[0]
    dst = edge_index[1]
    zi = jnp.zeros((npad,), jnp.int32)
    rowp = jnp.concatenate([row, zi])
    row3d = rowp.reshape(NW, nch, CG)
    et3d = jnp.concatenate([edge_type, zi]).reshape(NW, nch, CG)
    dst3d = jnp.concatenate([dst, zi]).reshape(NW, nch, CG)
    nchd = n_real // (NW * CGD)
    row3dd = row.reshape(NW, nchd, CGD)

    deg2 = _sc_deg(row3dd, n_ent)

    zpad = jnp.zeros((7, 128), F32)
    rp1 = jnp.concatenate([p['rel_info_att'], cpa['loop_rel'], zpad], axis=0)
    rp2 = jnp.concatenate([p['rel_completion_att'], cpc['loop_rel'], zpad],
                          axis=0)
    l3p = jnp.concatenate([cp2['loop_rel'], zpad], axis=0)
    e1, xi1, xic, rel1, rel2, rel3a, rel3b = _tc_prep0(
        p['ent_completion_att'], p['ent_info_att'],
        (p['align_linear1_1'][:128], p['align_linear1_1'][128:],
         p['align_linear1_2'], cpa['w_att'][:128], cpc['w_att'][:128]),
        (rp1, cpa['w1'], cpa['w2'], rp2, cpc['w1'], cpc['w2'],
         p['rel_info_att'], p['rel_linear11_align'], p['rel_linear12_align'],
         l3p, cp2['w1'], cp2['w2']))

    dinv128 = _tc_deginv(deg2[:n_ent].reshape(n_ent, 1),
                         deg2[n_ent:].reshape(n_ent, 1))
    norm16 = _sc_normg(dinv128, row3d, n_edge)
    norm1 = _tc_norm1(norm16, n_edge)

    hpa, suma, sqa = _edge_phase(cpa, e1, xi1, rel1, rel1[400:401],
                                 row3d, et3d, dst3d, norm1, n_ent, n_edge,
                                 n_real)
    hpc, sumc, sqc = _edge_phase(cpc, p['ent_completion_att'], xic, rel2,
                                 rel2[400:401], row3d, et3d, dst3d, norm1,
                                 n_ent, n_edge, n_real)

    a1, e2, xi2 = _tc_e2m(hpa, suma, sqa,
                          cpa['bn_g'].reshape(1, 128),
                          cpa['bn_b'].reshape(1, 128),
                          hpc, sumc, sqc,
                          cpc['bn_g'].reshape(1, 128),
                          cpc['bn_b'].reshape(1, 128),
                          p['align_linear2_1'][:128],
                          p['align_linear2_1'][128:],
                          p['align_linear2_2'], cp2['w_att'][:128])

    rel3 = jnp.concatenate([rel3a, rel3b], axis=0)
    hp2, sum2, sq2 = _edge_phase(cp2, e2, xi2, rel3, rel3b[0:1],
                                 row3d, et3d, dst3d, norm1, n_ent, n_edge,
                                 n_real)

    w = p['all_linear_comp']
    return _tc_final(hp2, sum2, sq2,
                     cp2['bn_g'].reshape(1, 128),
                     cp2['bn_b'].reshape(1, 128),
                     e1, a1, w[:128], w[128:256], w[256:384])


# speedup vs baseline: 1.9455x; 1.7732x over previous
import functools

import jax
import jax.numpy as jnp
from jax import lax
from jax.experimental import pallas as pl
from jax.experimental.pallas import tpu as pltpu
from jax.experimental.pallas import tpu_sc as plsc

SLOPE = 0.2
NC = 2
NS = 16
NW = NC * NS
CG = 80
CGD = 80
RB = 1000
EB = 2048
F32 = jnp.float32


def _lrelu(x):
    return jnp.where(x >= 0, x, SLOPE * x)


def _mesh():
    return plsc.VectorSubcoreMesh(core_axis_name="c", subcore_axis_name="s",
                                  num_cores=NC, num_subcores=NS)


def _sc_deg(row3d, n_ent):
    nch = row3d.shape[1]
    cg = row3d.shape[2]
    nz = n_ent // CGD

    @functools.partial(
        pl.kernel,
        out_type=jax.ShapeDtypeStruct((NC * n_ent,), F32),
        mesh=_mesh(),
        scratch_types=[
            pltpu.VMEM((nch, cg), jnp.int32),
            pltpu.VMEM((cg,), F32),
            pltpu.VMEM((CGD,), F32),
            pltpu.VMEM_SHARED((n_ent,), F32),
            pltpu.SemaphoreType.DMA,
        ],
    )
    def k(row_h, deg_h, row_v, ones_v, z1, dg_sh, dsem):
        ci = lax.axis_index("c")
        t = lax.axis_index("s")
        wid = t * NC + ci

        def fill(r, _):
            ones_v[pl.ds(r * 16, 16)] = jnp.full((16,), 1.0, F32)
            return _
        lax.fori_loop(0, cg // 16, fill, None)

        def fillz(r, _):
            z1[pl.ds(r * 16, 16)] = jnp.zeros((16,), F32)
            return _
        lax.fori_loop(0, CGD // 16, fillz, None)

        def zloop(j, _):
            cz = t + j * NS

            @pl.when(cz < nz)
            def _():
                pltpu.sync_copy(z1, dg_sh.at[pl.ds(cz * CGD, CGD)])
            return _
        lax.fori_loop(0, (nz + NS - 1) // NS, zloop, None)
        plsc.subcore_barrier()

        pltpu.sync_copy(row_h.at[wid], row_v)

        def fire(c, _):
            pltpu.async_copy(ones_v, dg_sh.at[row_v.at[c]], dsem, add=True)
            return _
        lax.fori_loop(0, nch, fire, None)

        def drain(c, _):
            pltpu.make_async_copy(ones_v, dg_sh.at[row_v.at[c]],
                                  dsem).wait()
            return _
        lax.fori_loop(0, nch, drain, None)

        plsc.subcore_barrier()

        def dloop(j, _):
            cz = t + j * NS

            @pl.when(cz < nz)
            def _():
                pltpu.sync_copy(dg_sh.at[pl.ds(cz * CGD, CGD)], z1)
                pltpu.sync_copy(z1,
                                deg_h.at[pl.ds(ci * n_ent + cz * CGD, CGD)])
            return _
        lax.fori_loop(0, (nz + NS - 1) // NS, dloop, None)

    return k(row3d)


def _sc_normg(dinv128, row3d, n_edge):
    nch = row3d.shape[1]

    @functools.partial(
        pl.kernel,
        out_type=jax.ShapeDtypeStruct((n_edge, 16), F32),
        mesh=_mesh(),
        scratch_types=[
            pltpu.VMEM((nch, CG), jnp.int32),
            pltpu.VMEM((2, CG, 128), F32),
            pltpu.VMEM((2, CG, 16), F32),
            pltpu.SemaphoreType.DMA((2,)),
            pltpu.SemaphoreType.DMA((2,)),
        ],
    )
    def k(dinv_h, row_h, out_h, row_v, nb, nb16, gsem, wsem):
        ci = lax.axis_index("c")
        t = lax.axis_index("s")
        wid = t * NC + ci
        pltpu.sync_copy(row_h.at[wid], row_v)

        def fire(c, s):
            pltpu.async_copy(dinv_h.at[row_v.at[c]], nb.at[s], gsem.at[s])

        def wait_g(c, s):
            pltpu.make_async_copy(dinv_h.at[row_v.at[c]], nb.at[s],
                                  gsem.at[s]).wait()

        def wait_w(c, s):
            off = pl.ds((wid * nch + c) * CG, CG)
            pltpu.make_async_copy(nb16.at[s], out_h.at[off],
                                  wsem.at[s]).wait()

        fire(0, 0)

        def step(c, s):
            wait_g(c, s)
            o = 1 - s

            @pl.when(c + 1 < nch)
            def _():
                fire(c + 1, o)

            @pl.when(c >= 2)
            def _():
                wait_w(c - 2, s)

            def ext(r, _):
                nb16[s, r, :] = nb[s, r, pl.ds(0, 16)]
                return _
            lax.fori_loop(0, CG, ext, None)
            off = pl.ds((wid * nch + c) * CG, CG)
            pltpu.async_copy(nb16.at[s], out_h.at[off], wsem.at[s])

        def body(c, _):
            @pl.when(c % 2 == 0)
            def _():
                step(c, 0)

            @pl.when(c % 2 == 1)
            def _():
                step(c, 1)
            return _
        lax.fori_loop(0, nch, body, None)
        wait_w(nch - 1, (nch - 1) % 2)
        wait_w(nch - 2, (nch - 2) % 2)

    return k(dinv128, row3d)


def _sc_gather(ent, rel_e, xi, row3d, et3d, dst3d, n_edge):
    nch = row3d.shape[1]

    @functools.partial(
        pl.kernel,
        out_type=[
            jax.ShapeDtypeStruct((n_edge, 128), F32),
            jax.ShapeDtypeStruct((n_edge, 128), F32),
        ],
        mesh=_mesh(),
        scratch_types=[
            pltpu.VMEM((nch, CG), jnp.int32),
            pltpu.VMEM((nch, CG), jnp.int32),
            pltpu.VMEM((nch, CG), jnp.int32),
            pltpu.VMEM((2, CG, 128), F32),
            pltpu.VMEM((2, CG, 128), F32),
            pltpu.VMEM((2, CG, 128), F32),
            pltpu.VMEM((2, CG, 128), F32),
            pltpu.SemaphoreType.DMA((2,)),
            pltpu.SemaphoreType.DMA((2,)),
        ],
    )
    def k(ent_h, rel_h, xi_h, row_h, et_h, dst_h, msg_h, xid_h,
          row_v, et_v, dst_v, xj_v, rl_v, xd_v, mo_v, gsem, wsem):
        ci = lax.axis_index("c")
        t = lax.axis_index("s")
        wid = t * NC + ci
        pltpu.sync_copy(row_h.at[wid], row_v)
        pltpu.sync_copy(et_h.at[wid], et_v)
        pltpu.sync_copy(dst_h.at[wid], dst_v)

        def fire(c, s):
            pltpu.async_copy(ent_h.at[row_v.at[c]], xj_v.at[s], gsem.at[s])
            pltpu.async_copy(rel_h.at[et_v.at[c]], rl_v.at[s], gsem.at[s])
            pltpu.async_copy(xi_h.at[dst_v.at[c]], xd_v.at[s], gsem.at[s])

        def wait_g(c, s):
            pltpu.make_async_copy(ent_h.at[row_v.at[c]], xj_v.at[s],
                                  gsem.at[s]).wait()
            pltpu.make_async_copy(rel_h.at[et_v.at[c]], rl_v.at[s],
                                  gsem.at[s]).wait()
            pltpu.make_async_copy(xi_h.at[dst_v.at[c]], xd_v.at[s],
                                  gsem.at[s]).wait()

        def wait_w(c, s):
            off = pl.ds((wid * nch + c) * CG, CG)
            pltpu.make_async_copy(mo_v.at[s], msg_h.at[off],
                                  wsem.at[s]).wait()
            pltpu.make_async_copy(xd_v.at[s], xid_h.at[off],
                                  wsem.at[s]).wait()

        fire(0, 0)

        def step(c, s):
            wait_g(c, s)
            o = 1 - s

            @pl.when(c + 1 < nch)
            def _():
                @pl.when(c >= 1)
                def _():
                    wait_w(c - 1, o)
                fire(c + 1, o)

            @plsc.parallel_loop(0, CG, step=1, unroll=4)
            def mul(r):
                for kk in range(8):
                    d = pl.ds(kk * 16, 16)
                    mo_v[s, r, d] = xj_v[s, r, d] * rl_v[s, r, d]

            off = pl.ds((wid * nch + c) * CG, CG)
            pltpu.async_copy(mo_v.at[s], msg_h.at[off], wsem.at[s])
            pltpu.async_copy(xd_v.at[s], xid_h.at[off], wsem.at[s])

        def body(c, _):
            @pl.when(c % 2 == 0)
            def _():
                step(c, 0)

            @pl.when(c % 2 == 1)
            def _():
                step(c, 1)
            return _
        lax.fori_loop(0, nch, body, None)
        wait_w(nch - 1, (nch - 1) % 2)
        wait_w(nch - 2, (nch - 2) % 2)

    return k(ent, rel_e, xi, row3d, et3d, dst3d)


def _sc_scatter(msgs, gg, dst3d, n_ent, n_edge):
    nch = dst3d.shape[1]
    nz = n_ent // CGD

    @functools.partial(
        pl.kernel,
        out_type=[
            jax.ShapeDtypeStruct((NC, n_ent, 128), F32),
            jax.ShapeDtypeStruct((NC * n_ent,), F32),
        ],
        mesh=_mesh(),
        scratch_types=[
            pltpu.VMEM((nch, CG), jnp.int32),
            pltpu.VMEM((2, CG, 128), F32),
            pltpu.VMEM((2, CG), F32),
            pltpu.VMEM_SHARED((n_ent, 128), F32),
            pltpu.VMEM_SHARED((n_ent,), F32),
            pltpu.SemaphoreType.DMA((2,)),
            pltpu.SemaphoreType.DMA((2,)),
        ],
    )
    def k(msgs_h, gg_h, dst_h, u_out, dn_out,
          dst_v, mb, gb, u_sh, dn_sh, rsem, ssem):
        ci = lax.axis_index("c")
        t = lax.axis_index("s")
        wid = t * NC + ci

        def fill_z(r, _):
            for kk in range(8):
                mb[0, r, pl.ds(kk * 16, 16)] = jnp.zeros((16,), F32)
            return _
        lax.fori_loop(0, CGD, fill_z, None)

        def fill_z1(r, _):
            gb[0, pl.ds(r * 16, 16)] = jnp.zeros((16,), F32)
            return _
        lax.fori_loop(0, CGD // 16, fill_z1, None)

        def zloop(j, _):
            cz = t + j * NS

            @pl.when(cz < nz)
            def _():
                pltpu.sync_copy(mb.at[0, pl.ds(0, CGD)],
                                u_sh.at[pl.ds(cz * CGD, CGD)])
                pltpu.sync_copy(gb.at[0, pl.ds(0, CGD)],
                                dn_sh.at[pl.ds(cz * CGD, CGD)])
            return _
        lax.fori_loop(0, (nz + NS - 1) // NS, zloop, None)
        plsc.subcore_barrier()

        pltpu.sync_copy(dst_h.at[wid], dst_v)

        def fire_r(c, s):
            off = pl.ds((wid * nch + c) * CG, CG)
            pltpu.async_copy(msgs_h.at[off], mb.at[s], rsem.at[s])
            pltpu.async_copy(gg_h.at[off], gb.at[s], rsem.at[s])

        def wait_r(c, s):
            off = pl.ds((wid * nch + c) * CG, CG)
            pltpu.make_async_copy(msgs_h.at[off], mb.at[s],
                                  rsem.at[s]).wait()
            pltpu.make_async_copy(gg_h.at[off], gb.at[s],
                                  rsem.at[s]).wait()

        def wait_s(c, s):
            pltpu.make_async_copy(mb.at[s], u_sh.at[dst_v.at[c]],
                                  ssem.at[s]).wait()
            pltpu.make_async_copy(gb.at[s], dn_sh.at[dst_v.at[c]],
                                  ssem.at[s]).wait()

        fire_r(0, 0)

        def step(c, s):
            wait_r(c, s)
            o = 1 - s

            @pl.when(c + 1 < nch)
            def _():
                @pl.when(c >= 1)
                def _():
                    wait_s(c - 1, o)
                fire_r(c + 1, o)

            pltpu.async_copy(mb.at[s], u_sh.at[dst_v.at[c]], ssem.at[s],
                             add=True)
            pltpu.async_copy(gb.at[s], dn_sh.at[dst_v.at[c]], ssem.at[s],
                             add=True)

        def body(c, _):
            @pl.when(c % 2 == 0)
            def _():
                step(c, 0)

            @pl.when(c % 2 == 1)
            def _():
                step(c, 1)
            return _
        lax.fori_loop(0, nch, body, None)
        wait_s(nch - 1, (nch - 1) % 2)
        wait_s(nch - 2, (nch - 2) % 2)

        plsc.subcore_barrier()

        def dloop(j, _):
            cz = t + j * NS

            @pl.when(cz < nz)
            def _():
                sl = pl.ds(cz * CGD, CGD)
                pltpu.sync_copy(u_sh.at[sl], mb.at[0, pl.ds(0, CGD)])
                pltpu.sync_copy(mb.at[0, pl.ds(0, CGD)], u_out.at[ci, sl])
                pltpu.sync_copy(dn_sh.at[sl], gb.at[0, pl.ds(0, CGD)])
                pltpu.sync_copy(gb.at[0, pl.ds(0, CGD)],
                                dn_out.at[pl.ds(ci * n_ent + cz * CGD, CGD)])
            return _
        lax.fori_loop(0, (nz + NS - 1) // NS, dloop, None)

    return k(msgs, gg, dst3d)


def _mm(a, b):
    return jnp.dot(a, b, preferred_element_type=F32)


def _l2n(a):
    nrm = jnp.sqrt(jnp.sum(a * a, axis=-1, keepdims=True))
    return a / jnp.maximum(nrm, 1e-12)


def _bn_tanh(hp, ssum, ssq, g, b, n):
    mean = ssum / n
    var = ssq / n - mean * mean
    inv = 1.0 / jnp.sqrt(var + 1e-5)
    return jnp.tanh((hp - mean) * inv * g + b)


def _wspec():
    return pl.BlockSpec((128, 128), lambda i: (0, 0))


def _rspec():
    return pl.BlockSpec((RB, 128), lambda i: (i, 0))


def _tc_prep0(ec, ei, lw, rel_args):
    n = ec.shape[0]
    nbk = n // RB
    (rp1, w11, w21, rp2, w12, w22, rinfo, r11a, r12a, l3p, w13, w23) = rel_args
    (l11a, l11b, l12, wtop1, wtopc) = lw

    def body(ec_ref, ei_ref, l11a_r, l11b_r, l12_r, wt1_r, wtc_r,
             rp1_r, w11_r, w21_r, rp2_r, w12_r, w22_r,
             ri_r, r11a_r, r12a_r, l3p_r, w13_r, w23_r,
             e1_ref, xi1_ref, xic_ref, rel1_ref, rel2_ref,
             rel3a_ref, rel3b_ref):
        a = _l2n(ec_ref[...])
        e1 = _mm(_lrelu(_mm(a, l11a_r[...]) + _mm(ei_ref[...], l11b_r[...])),
                 l12_r[...])
        e1_ref[...] = e1
        xi1_ref[...] = _mm(e1, wt1_r[...])
        xic_ref[...] = _mm(ec_ref[...], wtc_r[...])

        @pl.when(pl.program_id(0) == 0)
        def _():
            rel1_ref[...] = _mm(_lrelu(_mm(rp1_r[...], w11_r[...])),
                                w21_r[...])
            rel2_ref[...] = _mm(_lrelu(_mm(rp2_r[...], w12_r[...])),
                                w22_r[...])
            ar = _mm(_lrelu(_mm(ri_r[...], r11a_r[...])), r12a_r[...])
            rel3a_ref[...] = _mm(_lrelu(_mm(ar, w13_r[...])), w23_r[...])
            rel3b_ref[...] = _mm(_lrelu(_mm(l3p_r[...], w13_r[...])),
                                 w23_r[...])

    rel_spec = lambda shape: pl.BlockSpec(shape, lambda i: (0, 0))
    return pl.pallas_call(
        body,
        grid=(nbk,),
        in_specs=[_rspec(), _rspec(),
                  _wspec(), _wspec(), _wspec(), _wspec(), _wspec(),
                  rel_spec((408, 128)), _wspec(), _wspec(),
                  rel_spec((408, 128)), _wspec(), _wspec(),
                  rel_spec((400, 128)), _wspec(), _wspec(),
                  rel_spec((8, 128)), _wspec(), _wspec()],
        out_specs=[_rspec(), _rspec(), _rspec(),
                   rel_spec((408, 128)), rel_spec((408, 128)),
                   rel_spec((400, 128)), rel_spec((8, 128))],
        out_shape=[jax.ShapeDtypeStruct((n, 128), F32),
                   jax.ShapeDtypeStruct((n, 128), F32),
                   jax.ShapeDtypeStruct((n, 128), F32),
                   jax.ShapeDtypeStruct((408, 128), F32),
                   jax.ShapeDtypeStruct((408, 128), F32),
                   jax.ShapeDtypeStruct((400, 128), F32),
                   jax.ShapeDtypeStruct((8, 128), F32)],
    )(ec, ei, l11a, l11b, l12, wtop1, wtopc,
      rp1, w11, w21, rp2, w12, w22, rinfo, r11a, r12a, l3p, w13, w23)


def _tc_deginv(d0, d1):
    n = d0.shape[0]
    nbk = n // RB

    def body(d0_ref, d1_ref, o_ref):
        d = d0_ref[...] + d1_ref[...]
        o_ref[...] = jnp.broadcast_to(
            jnp.where(d > 0, 1.0 / jnp.sqrt(d), 0.0), (RB, 128))

    return pl.pallas_call(
        body,
        grid=(nbk,),
        in_specs=[pl.BlockSpec((RB, 1), lambda i: (i, 0)),
                  pl.BlockSpec((RB, 1), lambda i: (i, 0))],
        out_specs=pl.BlockSpec((RB, 128), lambda i: (i, 0)),
        out_shape=jax.ShapeDtypeStruct((n, 128), F32),
    )(d0, d1)


def _tc_norm1(norm16, n_edge):
    neb = n_edge // EB

    def body(n_ref, o_ref):
        o_ref[...] = n_ref[:, 0:1].reshape(EB)

    return pl.pallas_call(
        body,
        grid=(neb,),
        in_specs=[pl.BlockSpec((EB, 16), lambda i: (i, 0))],
        out_specs=pl.BlockSpec((EB,), lambda i: (i,)),
        out_shape=jax.ShapeDtypeStruct((n_edge,), F32),
    )(norm16)


def _tc_score_scale(msg, xid, wb, aa, norm1, n_edge, n_real):
    neb = n_edge // EB

    def body(msg_ref, xid_ref, wb_ref, aa_ref, nrm_ref, msgs_ref, gg_ref):
        z = _lrelu(_mm(msg_ref[...], wb_ref[...]) + xid_ref[...])
        s = _mm(z, aa_ref[...])
        i = pl.program_id(0)
        eidx = lax.broadcasted_iota(jnp.int32, (EB, 1), 0) + i * EB
        g = jnp.where(eidx < n_real, jnp.exp(s), 0.0)
        gg_ref[...] = g.reshape(EB)
        gn = g * nrm_ref[...].reshape(EB, 1)
        msgs_ref[...] = msg_ref[...] * gn

    return pl.pallas_call(
        body,
        grid=(neb,),
        in_specs=[pl.BlockSpec((EB, 128), lambda i: (i, 0)),
                  pl.BlockSpec((EB, 128), lambda i: (i, 0)),
                  pl.BlockSpec((128, 128), lambda i: (0, 0)),
                  pl.BlockSpec((128, 1), lambda i: (0, 0)),
                  pl.BlockSpec((EB,), lambda i: (i,))],
        out_specs=[pl.BlockSpec((EB, 128), lambda i: (i, 0)),
                   pl.BlockSpec((EB,), lambda i: (i,))],
        out_shape=[jax.ShapeDtypeStruct((n_edge, 128), F32),
                   jax.ShapeDtypeStruct((n_edge,), F32)],
    )(msg, xid, wb, aa, norm1)


def _tc_comb_a(u0, u1, d0, d1, ent, loopv, gcn):
    n = ent.shape[0]
    nbk = n // RB

    def body(u0_ref, u1_ref, d0_ref, d1_ref, ent_ref, lv_ref, g_ref,
             hp_ref, ssum_ref, ssq_ref):
        den = d0_ref[...] + d1_ref[...] + 1e-16
        pre = (u0_ref[...] + u1_ref[...]) / den + ent_ref[...] * lv_ref[...]
        hp = _mm(pre, g_ref[...]) * 0.5
        hp_ref[...] = hp
        cs = jnp.sum(hp, axis=0, keepdims=True)
        cq = jnp.sum(hp * hp, axis=0, keepdims=True)
        i = pl.program_id(0)

        @pl.when(i == 0)
        def _():
            ssum_ref[...] = cs
            ssq_ref[...] = cq

        @pl.when(i > 0)
        def _():
            ssum_ref[...] = ssum_ref[...] + cs
            ssq_ref[...] = ssq_ref[...] + cq

    vspec = pl.BlockSpec((1, 128), lambda i: (0, 0))
    return pl.pallas_call(
        body,
        grid=(nbk,),
        in_specs=[_rspec(), _rspec(),
                  pl.BlockSpec((RB, 1), lambda i: (i, 0)),
                  pl.BlockSpec((RB, 1), lambda i: (i, 0)),
                  _rspec(), vspec, _wspec()],
        out_specs=[_rspec(), vspec, vspec],
        out_shape=[jax.ShapeDtypeStruct((n, 128), F32),
                   jax.ShapeDtypeStruct((1, 128), F32),
                   jax.ShapeDtypeStruct((1, 128), F32)],
    )(u0, u1, d0, d1, ent, loopv, gcn)


def _tc_e2m(hpa, suma, sqa, bga, bba, hpc, sumc, sqc, bgc, bbc,
            l21a, l21b, l22, wtop2):
    n = hpa.shape[0]
    nbk = n // RB

    def body(hpa_r, sa_r, qa_r, ga_r, ba_r, hpc_r, sc_r, qc_r, gc_r, bc_r,
             l21a_r, l21b_r, l22_r, wt2_r, a1_ref, e2_ref, xi2_ref):
        a1 = _bn_tanh(hpa_r[...], sa_r[...], qa_r[...], ga_r[...],
                      ba_r[...], n)
        c1 = _bn_tanh(hpc_r[...], sc_r[...], qc_r[...], gc_r[...],
                      bc_r[...], n)
        a1_ref[...] = a1
        e2 = _mm(_lrelu(_mm(_l2n(c1), l21a_r[...]) + _mm(a1, l21b_r[...])),
                 l22_r[...])
        e2_ref[...] = e2
        xi2_ref[...] = _mm(e2, wt2_r[...])

    vspec = pl.BlockSpec((1, 128), lambda i: (0, 0))
    return pl.pallas_call(
        body,
        grid=(nbk,),
        in_specs=[_rspec(), vspec, vspec, vspec, vspec,
                  _rspec(), vspec, vspec, vspec, vspec,
                  _wspec(), _wspec(), _wspec(), _wspec()],
        out_specs=[_rspec(), _rspec(), _rspec()],
        out_shape=[jax.ShapeDtypeStruct((n, 128), F32),
                   jax.ShapeDtypeStruct((n, 128), F32),
                   jax.ShapeDtypeStruct((n, 128), F32)],
    )(hpa, suma, sqa, bga, bba, hpc, sumc, sqc, bgc, bbc,
      l21a, l21b, l22, wtop2)


def _tc_final(hp2, sum2, sq2, bg2, bb2, e1, a1, wa, wb, wc):
    n = hp2.shape[0]
    nbk = n // RB

    def body(hp_r, s_r, q_r, g_r, b_r, e1_r, a1_r, wa_r, wb_r, wc_r, o_ref):
        a2 = _bn_tanh(hp_r[...], s_r[...], q_r[...], g_r[...], b_r[...], n)
        o_ref[...] = (_mm(e1_r[...], wa_r[...]) + _mm(a1_r[...], wb_r[...])
                      + _mm(a2, wc_r[...]))

    vspec = pl.BlockSpec((1, 128), lambda i: (0, 0))
    return pl.pallas_call(
        body,
        grid=(nbk,),
        in_specs=[_rspec(), vspec, vspec, vspec, vspec,
                  _rspec(), _rspec(), _wspec(), _wspec(), _wspec()],
        out_specs=_rspec(),
        out_shape=jax.ShapeDtypeStruct((n, 128), F32),
    )(hp2, sum2, sq2, bg2, bb2, e1, a1, wa, wb, wc)


def _edge_phase(cp, ent, xi, rel_e, loopv, row3d, et3d, dst3d, norm1,
                n_ent, n_edge, n_real):
    msg, xid = _sc_gather(ent, rel_e, xi, row3d, et3d, dst3d, n_edge)
    msgs, gg = _tc_score_scale(msg, xid, cp['w_att'][128:], cp['a_att'],
                               norm1, n_edge, n_real)
    u2, dn2 = _sc_scatter(msgs, gg, dst3d, n_ent, n_edge)
    return _tc_comb_a(u2[0], u2[1],
                      dn2[:n_ent].reshape(n_ent, 1),
                      dn2[n_ent:].reshape(n_ent, 1),
                      ent, loopv, cp['gcn_w'])


def kernel(params, edge_index, edge_type):
    p = params
    cpa = p['conv1_align']
    cpc = p['conv1_completion']
    cp2 = p['conv2_align']
    n_ent = p['ent_completion_att'].shape[0]
    n_real = edge_type.shape[0]
    n_edge = ((n_real + NW * CG * 16 - 1) // (NW * CG * 16)) * (NW * CG * 16)
    nch = n_edge // (NW * CG)
    npad = n_edge - n_real

    row = edge_index[0]
    dst = edge_index[1]
    zi = jnp.arange(npad, dtype=jnp.int32)
    rowp = jnp.concatenate([row, (zi * 13) % n_ent])
    row3d = rowp.reshape(NW, nch, CG)
    et3d = jnp.concatenate([edge_type, zi % 400]).reshape(NW, nch, CG)
    dst3d = jnp.concatenate([dst, (zi * 29) % n_ent]).reshape(NW, nch, CG)
    nchd = n_real // (NW * CGD)
    row3dd = row.reshape(NW, nchd, CGD)

    deg2 = _sc_deg(row3dd, n_ent)

    zpad = jnp.zeros((7, 128), F32)
    rp1 = jnp.concatenate([p['rel_info_att'], cpa['loop_rel'], zpad], axis=0)
    rp2 = jnp.concatenate([p['rel_completion_att'], cpc['loop_rel'], zpad],
                          axis=0)
    l3p = jnp.concatenate([cp2['loop_rel'], zpad], axis=0)
    e1, xi1, xic, rel1, rel2, rel3a, rel3b = _tc_prep0(
        p['ent_completion_att'], p['ent_info_att'],
        (p['align_linear1_1'][:128], p['align_linear1_1'][128:],
         p['align_linear1_2'], cpa['w_att'][:128], cpc['w_att'][:128]),
        (rp1, cpa['w1'], cpa['w2'], rp2, cpc['w1'], cpc['w2'],
         p['rel_info_att'], p['rel_linear11_align'], p['rel_linear12_align'],
         l3p, cp2['w1'], cp2['w2']))

    dinv128 = _tc_deginv(deg2[:n_ent].reshape(n_ent, 1),
                         deg2[n_ent:].reshape(n_ent, 1))
    norm16 = _sc_normg(dinv128, row3d, n_edge)
    norm1 = _tc_norm1(norm16, n_edge)

    hpa, suma, sqa = _edge_phase(cpa, e1, xi1, rel1, rel1[400:401],
                                 row3d, et3d, dst3d, norm1, n_ent, n_edge,
                                 n_real)
    hpc, sumc, sqc = _edge_phase(cpc, p['ent_completion_att'], xic, rel2,
                                 rel2[400:401], row3d, et3d, dst3d, norm1,
                                 n_ent, n_edge, n_real)

    a1, e2, xi2 = _tc_e2m(hpa, suma, sqa,
                          cpa['bn_g'].reshape(1, 128),
                          cpa['bn_b'].reshape(1, 128),
                          hpc, sumc, sqc,
                          cpc['bn_g'].reshape(1, 128),
                          cpc['bn_b'].reshape(1, 128),
                          p['align_linear2_1'][:128],
                          p['align_linear2_1'][128:],
                          p['align_linear2_2'], cp2['w_att'][:128])

    rel3 = jnp.concatenate([rel3a, rel3b], axis=0)
    hp2, sum2, sq2 = _edge_phase(cp2, e2, xi2, rel3, rel3b[0:1],
                                 row3d, et3d, dst3d, norm1, n_ent, n_edge,
                                 n_real)

    w = p['all_linear_comp']
    return _tc_final(hp2, sum2, sq2,
                     cp2['bn_g'].reshape(1, 128),
                     cp2['bn_b'].reshape(1, 128),
                     e1, a1, w[:128], w[128:256], w[256:384])


# speedup vs baseline: 1.9669x; 1.0110x over previous
import functools

import jax
import jax.numpy as jnp
from jax import lax
from jax.experimental import pallas as pl
from jax.experimental.pallas import tpu as pltpu
from jax.experimental.pallas import tpu_sc as plsc

SLOPE = 0.2
NC = 2
NS = 16
NW = NC * NS
CG = 128
CGD = 80
RB = 1000
EB = 2048
F32 = jnp.float32


def _lrelu(x):
    return jnp.where(x >= 0, x, SLOPE * x)


def _mesh():
    return plsc.VectorSubcoreMesh(core_axis_name="c", subcore_axis_name="s",
                                  num_cores=NC, num_subcores=NS)


def _sc_deg(row3d, n_ent):
    nch = row3d.shape[1]
    cg = row3d.shape[2]
    nz = n_ent // CGD

    @functools.partial(
        pl.kernel,
        out_type=jax.ShapeDtypeStruct((NC * n_ent,), F32),
        mesh=_mesh(),
        scratch_types=[
            pltpu.VMEM((nch, cg), jnp.int32),
            pltpu.VMEM((cg,), F32),
            pltpu.VMEM((CGD,), F32),
            pltpu.VMEM_SHARED((n_ent,), F32),
            pltpu.SemaphoreType.DMA,
        ],
    )
    def k(row_h, deg_h, row_v, ones_v, z1, dg_sh, dsem):
        ci = lax.axis_index("c")
        t = lax.axis_index("s")
        wid = t * NC + ci

        def fill(r, _):
            ones_v[pl.ds(r * 16, 16)] = jnp.full((16,), 1.0, F32)
            return _
        lax.fori_loop(0, cg // 16, fill, None)

        def fillz(r, _):
            z1[pl.ds(r * 16, 16)] = jnp.zeros((16,), F32)
            return _
        lax.fori_loop(0, CGD // 16, fillz, None)

        def zloop(j, _):
            cz = t + j * NS

            @pl.when(cz < nz)
            def _():
                pltpu.sync_copy(z1, dg_sh.at[pl.ds(cz * CGD, CGD)])
            return _
        lax.fori_loop(0, (nz + NS - 1) // NS, zloop, None)
        plsc.subcore_barrier()

        pltpu.sync_copy(row_h.at[wid], row_v)

        def fire(c, _):
            pltpu.async_copy(ones_v, dg_sh.at[row_v.at[c]], dsem, add=True)
            return _
        lax.fori_loop(0, nch, fire, None)

        def drain(c, _):
            pltpu.make_async_copy(ones_v, dg_sh.at[row_v.at[c]],
                                  dsem).wait()
            return _
        lax.fori_loop(0, nch, drain, None)

        plsc.subcore_barrier()

        def dloop(j, _):
            cz = t + j * NS

            @pl.when(cz < nz)
            def _():
                pltpu.sync_copy(dg_sh.at[pl.ds(cz * CGD, CGD)], z1)
                pltpu.sync_copy(z1,
                                deg_h.at[pl.ds(ci * n_ent + cz * CGD, CGD)])
            return _
        lax.fori_loop(0, (nz + NS - 1) // NS, dloop, None)

    return k(row3d)


def _sc_normg(dinv128, row3d, n_edge):
    nch = row3d.shape[1]

    @functools.partial(
        pl.kernel,
        out_type=jax.ShapeDtypeStruct((n_edge, 16), F32),
        mesh=_mesh(),
        scratch_types=[
            pltpu.VMEM((nch, CG), jnp.int32),
            pltpu.VMEM((2, CG, 128), F32),
            pltpu.VMEM((2, CG, 16), F32),
            pltpu.SemaphoreType.DMA((2,)),
            pltpu.SemaphoreType.DMA((2,)),
        ],
    )
    def k(dinv_h, row_h, out_h, row_v, nb, nb16, gsem, wsem):
        ci = lax.axis_index("c")
        t = lax.axis_index("s")
        wid = t * NC + ci
        pltpu.sync_copy(row_h.at[wid], row_v)

        def fire(c, s):
            pltpu.async_copy(dinv_h.at[row_v.at[c]], nb.at[s], gsem.at[s])

        def wait_g(c, s):
            pltpu.make_async_copy(dinv_h.at[row_v.at[c]], nb.at[s],
                                  gsem.at[s]).wait()

        def wait_w(c, s):
            off = pl.ds((wid * nch + c) * CG, CG)
            pltpu.make_async_copy(nb16.at[s], out_h.at[off],
                                  wsem.at[s]).wait()

        fire(0, 0)

        def step(c, s):
            wait_g(c, s)
            o = 1 - s

            @pl.when(c + 1 < nch)
            def _():
                fire(c + 1, o)

            @pl.when(c >= 2)
            def _():
                wait_w(c - 2, s)

            def ext(r, _):
                nb16[s, r, :] = nb[s, r, pl.ds(0, 16)]
                return _
            lax.fori_loop(0, CG, ext, None)
            off = pl.ds((wid * nch + c) * CG, CG)
            pltpu.async_copy(nb16.at[s], out_h.at[off], wsem.at[s])

        def body(c, _):
            @pl.when(c % 2 == 0)
            def _():
                step(c, 0)

            @pl.when(c % 2 == 1)
            def _():
                step(c, 1)
            return _
        lax.fori_loop(0, nch, body, None)
        wait_w(nch - 1, (nch - 1) % 2)
        wait_w(nch - 2, (nch - 2) % 2)

    return k(dinv128, row3d)


def _sc_gather(ent, rel_e, xi, row3d, et3d, dst3d, n_edge):
    nch = row3d.shape[1]

    @functools.partial(
        pl.kernel,
        out_type=[
            jax.ShapeDtypeStruct((n_edge, 128), F32),
            jax.ShapeDtypeStruct((n_edge, 128), F32),
        ],
        mesh=_mesh(),
        scratch_types=[
            pltpu.VMEM((nch, CG), jnp.int32),
            pltpu.VMEM((nch, CG), jnp.int32),
            pltpu.VMEM((nch, CG), jnp.int32),
            pltpu.VMEM((2, CG, 128), F32),
            pltpu.VMEM((2, CG, 128), F32),
            pltpu.VMEM((2, CG, 128), F32),
            pltpu.SemaphoreType.DMA((2,)),
            pltpu.SemaphoreType.DMA((2,)),
        ],
    )
    def k(ent_h, rel_h, xi_h, row_h, et_h, dst_h, msg_h, xid_h,
          row_v, et_v, dst_v, xj_v, rl_v, xd_v, gsem, wsem):
        ci = lax.axis_index("c")
        t = lax.axis_index("s")
        wid = t * NC + ci
        pltpu.sync_copy(row_h.at[wid], row_v)
        pltpu.sync_copy(et_h.at[wid], et_v)
        pltpu.sync_copy(dst_h.at[wid], dst_v)

        def fire(c, s):
            pltpu.async_copy(ent_h.at[row_v.at[c]], xj_v.at[s], gsem.at[s])
            pltpu.async_copy(rel_h.at[et_v.at[c]], rl_v.at[s], gsem.at[s])
            pltpu.async_copy(xi_h.at[dst_v.at[c]], xd_v.at[s], gsem.at[s])

        def wait_g(c, s):
            pltpu.make_async_copy(ent_h.at[row_v.at[c]], xj_v.at[s],
                                  gsem.at[s]).wait()
            pltpu.make_async_copy(rel_h.at[et_v.at[c]], rl_v.at[s],
                                  gsem.at[s]).wait()
            pltpu.make_async_copy(xi_h.at[dst_v.at[c]], xd_v.at[s],
                                  gsem.at[s]).wait()

        def wait_w(c, s):
            off = pl.ds((wid * nch + c) * CG, CG)
            pltpu.make_async_copy(xj_v.at[s], msg_h.at[off],
                                  wsem.at[s]).wait()
            pltpu.make_async_copy(xd_v.at[s], xid_h.at[off],
                                  wsem.at[s]).wait()

        fire(0, 0)

        def step(c, s):
            wait_g(c, s)
            o = 1 - s

            @pl.when(c + 1 < nch)
            def _():
                @pl.when(c >= 1)
                def _():
                    wait_w(c - 1, o)
                fire(c + 1, o)

            @plsc.parallel_loop(0, CG, step=1, unroll=4)
            def mul(r):
                for kk in range(8):
                    d = pl.ds(kk * 16, 16)
                    xj_v[s, r, d] = xj_v[s, r, d] * rl_v[s, r, d]

            off = pl.ds((wid * nch + c) * CG, CG)
            pltpu.async_copy(xj_v.at[s], msg_h.at[off], wsem.at[s])
            pltpu.async_copy(xd_v.at[s], xid_h.at[off], wsem.at[s])

        def body(c, _):
            @pl.when(c % 2 == 0)
            def _():
                step(c, 0)

            @pl.when(c % 2 == 1)
            def _():
                step(c, 1)
            return _
        lax.fori_loop(0, nch, body, None)
        wait_w(nch - 1, (nch - 1) % 2)
        wait_w(nch - 2, (nch - 2) % 2)

    return k(ent, rel_e, xi, row3d, et3d, dst3d)


def _sc_scatter(msgs, gg, dst3d, n_ent, n_edge):
    nch = dst3d.shape[1]
    nz = n_ent // CGD

    @functools.partial(
        pl.kernel,
        out_type=[
            jax.ShapeDtypeStruct((NC, n_ent, 128), F32),
            jax.ShapeDtypeStruct((NC * n_ent,), F32),
        ],
        mesh=_mesh(),
        scratch_types=[
            pltpu.VMEM((nch, CG), jnp.int32),
            pltpu.VMEM((2, CG, 128), F32),
            pltpu.VMEM((2, CG), F32),
            pltpu.VMEM_SHARED((n_ent, 128), F32),
            pltpu.VMEM_SHARED((n_ent,), F32),
            pltpu.SemaphoreType.DMA((2,)),
            pltpu.SemaphoreType.DMA((2,)),
        ],
    )
    def k(msgs_h, gg_h, dst_h, u_out, dn_out,
          dst_v, mb, gb, u_sh, dn_sh, rsem, ssem):
        ci = lax.axis_index("c")
        t = lax.axis_index("s")
        wid = t * NC + ci

        def fill_z(r, _):
            for kk in range(8):
                mb[0, r, pl.ds(kk * 16, 16)] = jnp.zeros((16,), F32)
            return _
        lax.fori_loop(0, CGD, fill_z, None)

        def fill_z1(r, _):
            gb[0, pl.ds(r * 16, 16)] = jnp.zeros((16,), F32)
            return _
        lax.fori_loop(0, CGD // 16, fill_z1, None)

        def zloop(j, _):
            cz = t + j * NS

            @pl.when(cz < nz)
            def _():
                pltpu.sync_copy(mb.at[0, pl.ds(0, CGD)],
                                u_sh.at[pl.ds(cz * CGD, CGD)])
                pltpu.sync_copy(gb.at[0, pl.ds(0, CGD)],
                                dn_sh.at[pl.ds(cz * CGD, CGD)])
            return _
        lax.fori_loop(0, (nz + NS - 1) // NS, zloop, None)
        plsc.subcore_barrier()

        pltpu.sync_copy(dst_h.at[wid], dst_v)

        def fire_r(c, s):
            off = pl.ds((wid * nch + c) * CG, CG)
            pltpu.async_copy(msgs_h.at[off], mb.at[s], rsem.at[s])
            pltpu.async_copy(gg_h.at[off], gb.at[s], rsem.at[s])

        def wait_r(c, s):
            off = pl.ds((wid * nch + c) * CG, CG)
            pltpu.make_async_copy(msgs_h.at[off], mb.at[s],
                                  rsem.at[s]).wait()
            pltpu.make_async_copy(gg_h.at[off], gb.at[s],
                                  rsem.at[s]).wait()

        def wait_s(c, s):
            pltpu.make_async_copy(mb.at[s], u_sh.at[dst_v.at[c]],
                                  ssem.at[s]).wait()
            pltpu.make_async_copy(gb.at[s], dn_sh.at[dst_v.at[c]],
                                  ssem.at[s]).wait()

        fire_r(0, 0)

        def step(c, s):
            wait_r(c, s)
            o = 1 - s

            @pl.when(c + 1 < nch)
            def _():
                @pl.when(c >= 1)
                def _():
                    wait_s(c - 1, o)
                fire_r(c + 1, o)

            pltpu.async_copy(mb.at[s], u_sh.at[dst_v.at[c]], ssem.at[s],
                             add=True)
            pltpu.async_copy(gb.at[s], dn_sh.at[dst_v.at[c]], ssem.at[s],
                             add=True)

        def body(c, _):
            @pl.when(c % 2 == 0)
            def _():
                step(c, 0)

            @pl.when(c % 2 == 1)
            def _():
                step(c, 1)
            return _
        lax.fori_loop(0, nch, body, None)
        wait_s(nch - 1, (nch - 1) % 2)
        wait_s(nch - 2, (nch - 2) % 2)

        plsc.subcore_barrier()

        def dloop(j, _):
            cz = t + j * NS

            @pl.when(cz < nz)
            def _():
                sl = pl.ds(cz * CGD, CGD)
                pltpu.sync_copy(u_sh.at[sl], mb.at[0, pl.ds(0, CGD)])
                pltpu.sync_copy(mb.at[0, pl.ds(0, CGD)], u_out.at[ci, sl])
                pltpu.sync_copy(dn_sh.at[sl], gb.at[0, pl.ds(0, CGD)])
                pltpu.sync_copy(gb.at[0, pl.ds(0, CGD)],
                                dn_out.at[pl.ds(ci * n_ent + cz * CGD, CGD)])
            return _
        lax.fori_loop(0, (nz + NS - 1) // NS, dloop, None)

    return k(msgs, gg, dst3d)


def _mm(a, b):
    return jnp.dot(a, b, preferred_element_type=F32)


def _l2n(a):
    nrm = jnp.sqrt(jnp.sum(a * a, axis=-1, keepdims=True))
    return a / jnp.maximum(nrm, 1e-12)


def _bn_tanh(hp, ssum, ssq, g, b, n):
    mean = ssum / n
    var = ssq / n - mean * mean
    inv = 1.0 / jnp.sqrt(var + 1e-5)
    return jnp.tanh((hp - mean) * inv * g + b)


def _wspec():
    return pl.BlockSpec((128, 128), lambda i: (0, 0))


def _rspec():
    return pl.BlockSpec((RB, 128), lambda i: (i, 0))


def _tc_prep0(ec, ei, lw, rel_args):
    n = ec.shape[0]
    nbk = n // RB
    (rp1, w11, w21, rp2, w12, w22, rinfo, r11a, r12a, l3p, w13, w23) = rel_args
    (l11a, l11b, l12, wtop1, wtopc) = lw

    def body(ec_ref, ei_ref, l11a_r, l11b_r, l12_r, wt1_r, wtc_r,
             rp1_r, w11_r, w21_r, rp2_r, w12_r, w22_r,
             ri_r, r11a_r, r12a_r, l3p_r, w13_r, w23_r,
             e1_ref, xi1_ref, xic_ref, rel1_ref, rel2_ref,
             rel3a_ref, rel3b_ref):
        a = _l2n(ec_ref[...])
        e1 = _mm(_lrelu(_mm(a, l11a_r[...]) + _mm(ei_ref[...], l11b_r[...])),
                 l12_r[...])
        e1_ref[...] = e1
        xi1_ref[...] = _mm(e1, wt1_r[...])
        xic_ref[...] = _mm(ec_ref[...], wtc_r[...])

        @pl.when(pl.program_id(0) == 0)
        def _():
            rel1_ref[...] = _mm(_lrelu(_mm(rp1_r[...], w11_r[...])),
                                w21_r[...])
            rel2_ref[...] = _mm(_lrelu(_mm(rp2_r[...], w12_r[...])),
                                w22_r[...])
            ar = _mm(_lrelu(_mm(ri_r[...], r11a_r[...])), r12a_r[...])
            rel3a_ref[...] = _mm(_lrelu(_mm(ar, w13_r[...])), w23_r[...])
            rel3b_ref[...] = _mm(_lrelu(_mm(l3p_r[...], w13_r[...])),
                                 w23_r[...])

    rel_spec = lambda shape: pl.BlockSpec(shape, lambda i: (0, 0))
    return pl.pallas_call(
        body,
        grid=(nbk,),
        in_specs=[_rspec(), _rspec(),
                  _wspec(), _wspec(), _wspec(), _wspec(), _wspec(),
                  rel_spec((408, 128)), _wspec(), _wspec(),
                  rel_spec((408, 128)), _wspec(), _wspec(),
                  rel_spec((400, 128)), _wspec(), _wspec(),
                  rel_spec((8, 128)), _wspec(), _wspec()],
        out_specs=[_rspec(), _rspec(), _rspec(),
                   rel_spec((408, 128)), rel_spec((408, 128)),
                   rel_spec((400, 128)), rel_spec((8, 128))],
        out_shape=[jax.ShapeDtypeStruct((n, 128), F32),
                   jax.ShapeDtypeStruct((n, 128), F32),
                   jax.ShapeDtypeStruct((n, 128), F32),
                   jax.ShapeDtypeStruct((408, 128), F32),
                   jax.ShapeDtypeStruct((408, 128), F32),
                   jax.ShapeDtypeStruct((400, 128), F32),
                   jax.ShapeDtypeStruct((8, 128), F32)],
    )(ec, ei, l11a, l11b, l12, wtop1, wtopc,
      rp1, w11, w21, rp2, w12, w22, rinfo, r11a, r12a, l3p, w13, w23)


def _tc_deginv(d0, d1):
    n = d0.shape[0]
    nbk = n // RB

    def body(d0_ref, d1_ref, o_ref):
        d = d0_ref[...] + d1_ref[...]
        o_ref[...] = jnp.broadcast_to(
            jnp.where(d > 0, 1.0 / jnp.sqrt(d), 0.0), (RB, 128))

    return pl.pallas_call(
        body,
        grid=(nbk,),
        in_specs=[pl.BlockSpec((RB, 1), lambda i: (i, 0)),
                  pl.BlockSpec((RB, 1), lambda i: (i, 0))],
        out_specs=pl.BlockSpec((RB, 128), lambda i: (i, 0)),
        out_shape=jax.ShapeDtypeStruct((n, 128), F32),
    )(d0, d1)


def _tc_norm1(norm16, n_edge):
    neb = n_edge // EB

    def body(n_ref, o_ref):
        o_ref[...] = n_ref[:, 0:1].reshape(EB)

    return pl.pallas_call(
        body,
        grid=(neb,),
        in_specs=[pl.BlockSpec((EB, 16), lambda i: (i, 0))],
        out_specs=pl.BlockSpec((EB,), lambda i: (i,)),
        out_shape=jax.ShapeDtypeStruct((n_edge,), F32),
    )(norm16)


def _tc_score_scale(msg, xid, wb, aa, norm1, n_edge, n_real):
    neb = n_edge // EB

    def body(msg_ref, xid_ref, wb_ref, aa_ref, nrm_ref, msgs_ref, gg_ref):
        z = _lrelu(_mm(msg_ref[...], wb_ref[...]) + xid_ref[...])
        s = _mm(z, aa_ref[...])
        i = pl.program_id(0)
        eidx = lax.broadcasted_iota(jnp.int32, (EB, 1), 0) + i * EB
        g = jnp.where(eidx < n_real, jnp.exp(s), 0.0)
        gg_ref[...] = g.reshape(EB)
        gn = g * nrm_ref[...].reshape(EB, 1)
        msgs_ref[...] = msg_ref[...] * gn

    return pl.pallas_call(
        body,
        grid=(neb,),
        in_specs=[pl.BlockSpec((EB, 128), lambda i: (i, 0)),
                  pl.BlockSpec((EB, 128), lambda i: (i, 0)),
                  pl.BlockSpec((128, 128), lambda i: (0, 0)),
                  pl.BlockSpec((128, 1), lambda i: (0, 0)),
                  pl.BlockSpec((EB,), lambda i: (i,))],
        out_specs=[pl.BlockSpec((EB, 128), lambda i: (i, 0)),
                   pl.BlockSpec((EB,), lambda i: (i,))],
        out_shape=[jax.ShapeDtypeStruct((n_edge, 128), F32),
                   jax.ShapeDtypeStruct((n_edge,), F32)],
    )(msg, xid, wb, aa, norm1)


def _tc_comb_a(u0, u1, d0, d1, ent, loopv, gcn):
    n = ent.shape[0]
    nbk = n // RB

    def body(u0_ref, u1_ref, d0_ref, d1_ref, ent_ref, lv_ref, g_ref,
             hp_ref, ssum_ref, ssq_ref):
        den = d0_ref[...] + d1_ref[...] + 1e-16
        pre = (u0_ref[...] + u1_ref[...]) / den + ent_ref[...] * lv_ref[...]
        hp = _mm(pre, g_ref[...]) * 0.5
        hp_ref[...] = hp
        cs = jnp.sum(hp, axis=0, keepdims=True)
        cq = jnp.sum(hp * hp, axis=0, keepdims=True)
        i = pl.program_id(0)

        @pl.when(i == 0)
        def _():
            ssum_ref[...] = cs
            ssq_ref[...] = cq

        @pl.when(i > 0)
        def _():
            ssum_ref[...] = ssum_ref[...] + cs
            ssq_ref[...] = ssq_ref[...] + cq

    vspec = pl.BlockSpec((1, 128), lambda i: (0, 0))
    return pl.pallas_call(
        body,
        grid=(nbk,),
        in_specs=[_rspec(), _rspec(),
                  pl.BlockSpec((RB, 1), lambda i: (i, 0)),
                  pl.BlockSpec((RB, 1), lambda i: (i, 0)),
                  _rspec(), vspec, _wspec()],
        out_specs=[_rspec(), vspec, vspec],
        out_shape=[jax.ShapeDtypeStruct((n, 128), F32),
                   jax.ShapeDtypeStruct((1, 128), F32),
                   jax.ShapeDtypeStruct((1, 128), F32)],
    )(u0, u1, d0, d1, ent, loopv, gcn)


def _tc_e2m(hpa, suma, sqa, bga, bba, hpc, sumc, sqc, bgc, bbc,
            l21a, l21b, l22, wtop2):
    n = hpa.shape[0]
    nbk = n // RB

    def body(hpa_r, sa_r, qa_r, ga_r, ba_r, hpc_r, sc_r, qc_r, gc_r, bc_r,
             l21a_r, l21b_r, l22_r, wt2_r, a1_ref, e2_ref, xi2_ref):
        a1 = _bn_tanh(hpa_r[...], sa_r[...], qa_r[...], ga_r[...],
                      ba_r[...], n)
        c1 = _bn_tanh(hpc_r[...], sc_r[...], qc_r[...], gc_r[...],
                      bc_r[...], n)
        a1_ref[...] = a1
        e2 = _mm(_lrelu(_mm(_l2n(c1), l21a_r[...]) + _mm(a1, l21b_r[...])),
                 l22_r[...])
        e2_ref[...] = e2
        xi2_ref[...] = _mm(e2, wt2_r[...])

    vspec = pl.BlockSpec((1, 128), lambda i: (0, 0))
    return pl.pallas_call(
        body,
        grid=(nbk,),
        in_specs=[_rspec(), vspec, vspec, vspec, vspec,
                  _rspec(), vspec, vspec, vspec, vspec,
                  _wspec(), _wspec(), _wspec(), _wspec()],
        out_specs=[_rspec(), _rspec(), _rspec()],
        out_shape=[jax.ShapeDtypeStruct((n, 128), F32),
                   jax.ShapeDtypeStruct((n, 128), F32),
                   jax.ShapeDtypeStruct((n, 128), F32)],
    )(hpa, suma, sqa, bga, bba, hpc, sumc, sqc, bgc, bbc,
      l21a, l21b, l22, wtop2)


def _tc_final(hp2, sum2, sq2, bg2, bb2, e1, a1, wa, wb, wc):
    n = hp2.shape[0]
    nbk = n // RB

    def body(hp_r, s_r, q_r, g_r, b_r, e1_r, a1_r, wa_r, wb_r, wc_r, o_ref):
        a2 = _bn_tanh(hp_r[...], s_r[...], q_r[...], g_r[...], b_r[...], n)
        o_ref[...] = (_mm(e1_r[...], wa_r[...]) + _mm(a1_r[...], wb_r[...])
                      + _mm(a2, wc_r[...]))

    vspec = pl.BlockSpec((1, 128), lambda i: (0, 0))
    return pl.pallas_call(
        body,
        grid=(nbk,),
        in_specs=[_rspec(), vspec, vspec, vspec, vspec,
                  _rspec(), _rspec(), _wspec(), _wspec(), _wspec()],
        out_specs=_rspec(),
        out_shape=jax.ShapeDtypeStruct((n, 128), F32),
    )(hp2, sum2, sq2, bg2, bb2, e1, a1, wa, wb, wc)


def _edge_phase(cp, ent, xi, rel_e, loopv, row3d, et3d, dst3d, norm1,
                n_ent, n_edge, n_real):
    msg, xid = _sc_gather(ent, rel_e, xi, row3d, et3d, dst3d, n_edge)
    msgs, gg = _tc_score_scale(msg, xid, cp['w_att'][128:], cp['a_att'],
                               norm1, n_edge, n_real)
    u2, dn2 = _sc_scatter(msgs, gg, dst3d, n_ent, n_edge)
    return _tc_comb_a(u2[0], u2[1],
                      dn2[:n_ent].reshape(n_ent, 1),
                      dn2[n_ent:].reshape(n_ent, 1),
                      ent, loopv, cp['gcn_w'])


def kernel(params, edge_index, edge_type):
    p = params
    cpa = p['conv1_align']
    cpc = p['conv1_completion']
    cp2 = p['conv2_align']
    n_ent = p['ent_completion_att'].shape[0]
    n_real = edge_type.shape[0]
    n_edge = ((n_real + NW * CG * 16 - 1) // (NW * CG * 16)) * (NW * CG * 16)
    nch = n_edge // (NW * CG)
    npad = n_edge - n_real

    row = edge_index[0]
    dst = edge_index[1]
    zi = jnp.arange(npad, dtype=jnp.int32)
    rowp = jnp.concatenate([row, (zi * 13) % n_ent])
    row3d = rowp.reshape(NW, nch, CG)
    et3d = jnp.concatenate([edge_type, zi % 400]).reshape(NW, nch, CG)
    dst3d = jnp.concatenate([dst, (zi * 29) % n_ent]).reshape(NW, nch, CG)
    nchd = n_real // (NW * CGD)
    row3dd = row.reshape(NW, nchd, CGD)

    deg2 = _sc_deg(row3dd, n_ent)

    zpad = jnp.zeros((7, 128), F32)
    rp1 = jnp.concatenate([p['rel_info_att'], cpa['loop_rel'], zpad], axis=0)
    rp2 = jnp.concatenate([p['rel_completion_att'], cpc['loop_rel'], zpad],
                          axis=0)
    l3p = jnp.concatenate([cp2['loop_rel'], zpad], axis=0)
    e1, xi1, xic, rel1, rel2, rel3a, rel3b = _tc_prep0(
        p['ent_completion_att'], p['ent_info_att'],
        (p['align_linear1_1'][:128], p['align_linear1_1'][128:],
         p['align_linear1_2'], cpa['w_att'][:128], cpc['w_att'][:128]),
        (rp1, cpa['w1'], cpa['w2'], rp2, cpc['w1'], cpc['w2'],
         p['rel_info_att'], p['rel_linear11_align'], p['rel_linear12_align'],
         l3p, cp2['w1'], cp2['w2']))

    dinv128 = _tc_deginv(deg2[:n_ent].reshape(n_ent, 1),
                         deg2[n_ent:].reshape(n_ent, 1))
    norm16 = _sc_normg(dinv128, row3d, n_edge)
    norm1 = _tc_norm1(norm16, n_edge)

    hpa, suma, sqa = _edge_phase(cpa, e1, xi1, rel1, rel1[400:401],
                                 row3d, et3d, dst3d, norm1, n_ent, n_edge,
                                 n_real)
    hpc, sumc, sqc = _edge_phase(cpc, p['ent_completion_att'], xic, rel2,
                                 rel2[400:401], row3d, et3d, dst3d, norm1,
                                 n_ent, n_edge, n_real)

    a1, e2, xi2 = _tc_e2m(hpa, suma, sqa,
                          cpa['bn_g'].reshape(1, 128),
                          cpa['bn_b'].reshape(1, 128),
                          hpc, sumc, sqc,
                          cpc['bn_g'].reshape(1, 128),
                          cpc['bn_b'].reshape(1, 128),
                          p['align_linear2_1'][:128],
                          p['align_linear2_1'][128:],
                          p['align_linear2_2'], cp2['w_att'][:128])

    rel3 = jnp.concatenate([rel3a, rel3b], axis=0)
    hp2, sum2, sq2 = _edge_phase(cp2, e2, xi2, rel3, rel3b[0:1],
                                 row3d, et3d, dst3d, norm1, n_ent, n_edge,
                                 n_real)

    w = p['all_linear_comp']
    return _tc_final(hp2, sum2, sq2,
                     cp2['bn_g'].reshape(1, 128),
                     cp2['bn_b'].reshape(1, 128),
                     e1, a1, w[:128], w[128:256], w[256:384])
